# speedup vs baseline: 5.1617x; 5.1617x over previous
import functools

import jax
import jax.numpy as jnp
from jax.experimental import pallas as pl
from jax.experimental.pallas import tpu as pltpu

N = 10000
E = 160000
NC = 9
BB_C = 32
N_BB = 3
TOT_C = 35
HEADS = 8
VAL_C = 8
EDGE_C = 64

EBLK = 4000


def _edge_pre_kernel(ef_ref, we1_ref, wae_ref, weue_ref, ea_ref, eeu_ref):
    ef = ef_ref[...]
    emb = jax.nn.silu(ef @ we1_ref[...])
    ea_ref[...] = emb @ wae_ref[...]
    eeu_ref[...] = ef @ weue_ref[...]


def _edge_pre(edge_features, W_e1, Wa_e, Weu_e):
    grid = (E // EBLK,)
    return pl.pallas_call(
        _edge_pre_kernel,
        grid=grid,
        in_specs=[
            pl.BlockSpec((EBLK, EDGE_C), lambda i: (i, 0)),
            pl.BlockSpec((EDGE_C, EDGE_C), lambda i: (0, 0)),
            pl.BlockSpec((EDGE_C, HEADS), lambda i: (0, 0)),
            pl.BlockSpec((EDGE_C, EDGE_C), lambda i: (0, 0)),
        ],
        out_specs=[
            pl.BlockSpec((EBLK, HEADS), lambda i: (i, 0)),
            pl.BlockSpec((EBLK, EDGE_C), lambda i: (i, 0)),
        ],
        out_shape=[
            jax.ShapeDtypeStruct((E, HEADS), jnp.float32),
            jax.ShapeDtypeStruct((E, EDGE_C), jnp.float32),
        ],
    )(edge_features, W_e1, Wa_e, Weu_e)


def _edge_final_kernel(s_ref, eeu_ref, weu2_ref, out_ref):
    h = jax.nn.silu(s_ref[...] + eeu_ref[...])
    out_ref[...] = h @ weu2_ref[...]


def _edge_final(s, eeu, W_eu2):
    grid = (E // EBLK,)
    return pl.pallas_call(
        _edge_final_kernel,
        grid=grid,
        in_specs=[
            pl.BlockSpec((EBLK, EDGE_C), lambda i: (i, 0)),
            pl.BlockSpec((EBLK, EDGE_C), lambda i: (i, 0)),
            pl.BlockSpec((EDGE_C, EDGE_C), lambda i: (0, 0)),
        ],
        out_specs=pl.BlockSpec((EBLK, EDGE_C), lambda i: (i, 0)),
        out_shape=jax.ShapeDtypeStruct((E, EDGE_C), jnp.float32),
    )(s, eeu, W_eu2)


def kernel(bb_rel, bb_features, edge_features, edge_index, noising_mask,
           W_e1, W_alpha, W_v, W_proj, W_g, W_ff, W_eu1, W_eu2):
    src, dst = edge_index[0], edge_index[1]
    mask_f = noising_mask.astype(jnp.float32)

    x0 = jnp.concatenate(
        [bb_features[:, 0, :], jnp.zeros((N, 2), jnp.float32), mask_f[:, None]],
        axis=-1)
    a_src = x0 @ W_alpha[:TOT_C]
    a_dst = x0 @ W_alpha[TOT_C:2 * TOT_C]
    x = jnp.zeros((N, NC, TOT_C), jnp.float32)
    x = x.at[..., :BB_C].set(bb_features)
    x = x.at[:, 1:4, BB_C:].set(jnp.swapaxes(bb_rel, -1, -2))
    x = x.at[:, 0, TOT_C - 1].set(mask_f)
    v_node = jnp.einsum('nkc,chd->nkhd', x, W_v).reshape(N, NC * HEADS * VAL_C)

    Wa_e = W_alpha[2 * TOT_C:]
    Weu_e = W_eu1[2 * BB_C:]
    e_alpha, e_eu = _edge_pre(edge_features, W_e1, Wa_e, Weu_e)

    logits = a_src[src] + a_dst[dst] + e_alpha
    logits = jnp.where(logits >= 0, logits, 0.2 * logits)
    ex = jnp.exp(logits)
    denom = jax.ops.segment_sum(ex, dst, num_segments=N)
    alpha = ex / (denom[dst] + 1e-9)

    av = v_node[src].reshape(E, NC, HEADS, VAL_C) * alpha[:, None, :, None]
    u = jax.ops.segment_sum(av.reshape(E, NC * HEADS * VAL_C), dst,
                            num_segments=N)
    agg = u.reshape(N, NC, HEADS * VAL_C) @ W_proj

    gate = jax.nn.silu(agg[:, 0:1, :] @ W_g)
    new_bb = agg + (agg @ W_ff) * gate

    nb0 = new_bb[:, 0, :]
    b_src = nb0 @ W_eu1[:BB_C]
    b_dst = nb0 @ W_eu1[BB_C:2 * BB_C]
    s = b_src[src] + b_dst[dst]
    new_edge = _edge_final(s, e_eu, W_eu2)
    return new_bb, new_edge


# speedup vs baseline: 5.6915x; 1.1026x over previous
import functools

import jax
import jax.numpy as jnp
from jax import lax
from jax.experimental import pallas as pl
from jax.experimental.pallas import tpu as pltpu
from jax.experimental.pallas import tpu_sc as plsc

N = 10000
E = 160000
NC = 9
BB_C = 32
N_BB = 3
TOT_C = 35
HEADS = 8
VAL_C = 8
EDGE_C = 64

EBLK = 4000


def _edge_pre_kernel(ef_ref, we1_ref, wae_ref, weue_ref, ea_ref, eeu_ref):
    ef = ef_ref[...]
    emb = jax.nn.silu(ef @ we1_ref[...])
    ea_ref[...] = emb @ wae_ref[...]
    eeu_ref[...] = ef @ weue_ref[...]


def _edge_pre(edge_features, W_e1, Wa_e, Weu_e):
    grid = (E // EBLK,)
    return pl.pallas_call(
        _edge_pre_kernel,
        grid=grid,
        in_specs=[
            pl.BlockSpec((EBLK, EDGE_C), lambda i: (i, 0)),
            pl.BlockSpec((EDGE_C, EDGE_C), lambda i: (0, 0)),
            pl.BlockSpec((EDGE_C, HEADS), lambda i: (0, 0)),
            pl.BlockSpec((EDGE_C, EDGE_C), lambda i: (0, 0)),
        ],
        out_specs=[
            pl.BlockSpec((EBLK, HEADS), lambda i: (i, 0)),
            pl.BlockSpec((EBLK, EDGE_C), lambda i: (i, 0)),
        ],
        out_shape=[
            jax.ShapeDtypeStruct((E, HEADS), jnp.float32),
            jax.ShapeDtypeStruct((E, EDGE_C), jnp.float32),
        ],
    )(edge_features, W_e1, Wa_e, Weu_e)


def _edge_final_kernel(h_ref, weu2_ref, out_ref):
    h = jax.nn.silu(h_ref[...])
    out_ref[...] = h @ weu2_ref[...]


def _edge_final(h_pre, W_eu2):
    grid = (E // EBLK,)
    return pl.pallas_call(
        _edge_final_kernel,
        grid=grid,
        in_specs=[
            pl.BlockSpec((EBLK, EDGE_C), lambda i: (i, 0)),
            pl.BlockSpec((EDGE_C, EDGE_C), lambda i: (0, 0)),
        ],
        out_specs=pl.BlockSpec((EBLK, EDGE_C), lambda i: (i, 0)),
        out_shape=jax.ShapeDtypeStruct((E, EDGE_C), jnp.float32),
    )(h_pre, W_eu2)


_SC_NC = 2
_SC_NS = 16
_NW = _SC_NC * _SC_NS
_CH = 125
_CPT = E // (_NW * _CH)
_SC_MESH = dict(core_axis_name="c", subcore_axis_name="s")


def _scd_body(srcc, dstc, bsrc, bdst, eeuc, out,
              sidx, didx, bs, bd, eb, ob, sem):
    c = lax.axis_index("c")
    s = lax.axis_index("s")
    wid = s * _SC_NC + c
    row0 = wid * _CPT

    def chunk(ch, carry):
        row = row0 + ch
        pltpu.sync_copy(srcc.at[pl.ds(row, 1)], sidx)
        pltpu.sync_copy(dstc.at[pl.ds(row, 1)], didx)
        pltpu.sync_copy(eeuc.at[pl.ds(row, 1)], eb)
        pltpu.async_copy(bsrc.at[sidx.at[0]], bs, sem).wait()
        pltpu.async_copy(bdst.at[didx.at[0]], bd, sem).wait()

        @plsc.parallel_loop(0, _CH, 1, unroll=4)
        def _(r):
            for q in range(EDGE_C // 16):
                sl = pl.ds(q * 16, 16)
                ob[0, r, sl] = bs[r, sl] + bd[r, sl] + eb[0, r, sl]

        pltpu.sync_copy(ob, out.at[pl.ds(row, 1)])
        return carry

    lax.fori_loop(0, _CPT, chunk, 0)


def _scd_call(srcc, dstc, b_src, b_dst, eeuc):
    f = pl.kernel(
        _scd_body,
        out_type=jax.ShapeDtypeStruct((_NW * _CPT, _CH, EDGE_C), jnp.float32),
        mesh=plsc.VectorSubcoreMesh(**_SC_MESH),
        compiler_params=pltpu.CompilerParams(use_tc_tiling_on_sc=False),
        scratch_types=[
            pltpu.VMEM((1, _CH), jnp.int32),
            pltpu.VMEM((1, _CH), jnp.int32),
            pltpu.VMEM((_CH, EDGE_C), jnp.float32),
            pltpu.VMEM((_CH, EDGE_C), jnp.float32),
            pltpu.VMEM((1, _CH, EDGE_C), jnp.float32),
            pltpu.VMEM((1, _CH, EDGE_C), jnp.float32),
            pltpu.SemaphoreType.DMA,
        ],
    )
    return f(srcc, dstc, b_src, b_dst, eeuc)


def kernel(bb_rel, bb_features, edge_features, edge_index, noising_mask,
           W_e1, W_alpha, W_v, W_proj, W_g, W_ff, W_eu1, W_eu2):
    src, dst = edge_index[0], edge_index[1]
    mask_f = noising_mask.astype(jnp.float32)

    x0 = jnp.concatenate(
        [bb_features[:, 0, :], jnp.zeros((N, 2), jnp.float32), mask_f[:, None]],
        axis=-1)
    a_src = x0 @ W_alpha[:TOT_C]
    a_dst = x0 @ W_alpha[TOT_C:2 * TOT_C]
    x = jnp.zeros((N, NC, TOT_C), jnp.float32)
    x = x.at[..., :BB_C].set(bb_features)
    x = x.at[:, 1:4, BB_C:].set(jnp.swapaxes(bb_rel, -1, -2))
    x = x.at[:, 0, TOT_C - 1].set(mask_f)
    v_node = jnp.einsum('nkc,chd->nkhd', x, W_v).reshape(N, NC * HEADS * VAL_C)

    Wa_e = W_alpha[2 * TOT_C:]
    Weu_e = W_eu1[2 * BB_C:]
    e_alpha, e_eu = _edge_pre(edge_features, W_e1, Wa_e, Weu_e)

    logits = a_src[src] + a_dst[dst] + e_alpha
    logits = jnp.where(logits >= 0, logits, 0.2 * logits)
    ex = jnp.exp(logits)
    denom = jax.ops.segment_sum(ex, dst, num_segments=N)
    alpha = ex / (denom[dst] + 1e-9)

    av = v_node[src].reshape(E, NC, HEADS, VAL_C) * alpha[:, None, :, None]
    u = jax.ops.segment_sum(av.reshape(E, NC * HEADS * VAL_C), dst,
                            num_segments=N)
    agg = u.reshape(N, NC, HEADS * VAL_C) @ W_proj

    gate = jax.nn.silu(agg[:, 0:1, :] @ W_g)
    new_bb = agg + (agg @ W_ff) * gate

    nb0 = new_bb[:, 0, :]
    b_src = nb0 @ W_eu1[:BB_C]
    b_dst = nb0 @ W_eu1[BB_C:2 * BB_C]
    srcc = src.reshape(_NW * _CPT, _CH)
    dstc = dst.reshape(_NW * _CPT, _CH)
    eeuc = e_eu.reshape(_NW * _CPT, _CH, EDGE_C)
    h_pre = _scd_call(srcc, dstc, b_src, b_dst, eeuc)
    new_edge = _edge_final(h_pre.reshape(E, EDGE_C), W_eu2)
    return new_bb, new_edge


# speedup vs baseline: 10.7754x; 1.8932x over previous
import functools

import jax
import jax.numpy as jnp
from jax import lax
from jax.experimental import pallas as pl
from jax.experimental.pallas import tpu as pltpu
from jax.experimental.pallas import tpu_sc as plsc

N = 10000
E = 160000
NC = 9
BB_C = 32
N_BB = 3
TOT_C = 35
HEADS = 8
VAL_C = 8
EDGE_C = 64

EBLK = 4000


def _edge_pre_kernel(ef_ref, we1_ref, wae_ref, weue_ref, ea_ref, eeu_ref):
    ef = ef_ref[...]
    emb = jax.nn.silu(ef @ we1_ref[...])
    ea_ref[...] = emb @ wae_ref[...]
    eeu_ref[...] = ef @ weue_ref[...]


def _edge_pre(edge_features, W_e1, Wa_e16, Weu_e):
    grid = (E // EBLK,)
    return pl.pallas_call(
        _edge_pre_kernel,
        grid=grid,
        in_specs=[
            pl.BlockSpec((EBLK, EDGE_C), lambda i: (i, 0)),
            pl.BlockSpec((EDGE_C, EDGE_C), lambda i: (0, 0)),
            pl.BlockSpec((EDGE_C, 2 * HEADS), lambda i: (0, 0)),
            pl.BlockSpec((EDGE_C, EDGE_C), lambda i: (0, 0)),
        ],
        out_specs=[
            pl.BlockSpec((EBLK, 2 * HEADS), lambda i: (i, 0)),
            pl.BlockSpec((EBLK, EDGE_C), lambda i: (i, 0)),
        ],
        out_shape=[
            jax.ShapeDtypeStruct((E, 2 * HEADS), jnp.float32),
            jax.ShapeDtypeStruct((E, EDGE_C), jnp.float32),
        ],
    )(edge_features, W_e1, Wa_e16, Weu_e)


def _edge_final_kernel(h_ref, weu2_ref, out_ref):
    h = jax.nn.silu(h_ref[...])
    out_ref[...] = h @ weu2_ref[...]


def _edge_final(h_pre, W_eu2):
    grid = (E // EBLK,)
    return pl.pallas_call(
        _edge_final_kernel,
        grid=grid,
        in_specs=[
            pl.BlockSpec((EBLK, EDGE_C), lambda i: (i, 0)),
            pl.BlockSpec((EDGE_C, EDGE_C), lambda i: (0, 0)),
        ],
        out_specs=pl.BlockSpec((EBLK, EDGE_C), lambda i: (i, 0)),
        out_shape=jax.ShapeDtypeStruct((E, EDGE_C), jnp.float32),
    )(h_pre, W_eu2)


_SC_NC = 2
_SC_NS = 16
_NW = _SC_NC * _SC_NS
_CH = 125
_CPT = E // (_NW * _CH)
_SC_MESH = dict(core_axis_name="c", subcore_axis_name="s")


def _scd_body(srcc, dstc, bsrc, bdst, eeuc, out,
              sidx, didx, bs, bd, eb, ob, sem):
    c = lax.axis_index("c")
    s = lax.axis_index("s")
    wid = s * _SC_NC + c
    row0 = wid * _CPT

    def chunk(ch, carry):
        row = row0 + ch
        pltpu.sync_copy(srcc.at[pl.ds(row, 1)], sidx)
        pltpu.sync_copy(dstc.at[pl.ds(row, 1)], didx)
        pltpu.sync_copy(eeuc.at[pl.ds(row, 1)], eb)
        pltpu.async_copy(bsrc.at[sidx.at[0]], bs, sem).wait()
        pltpu.async_copy(bdst.at[didx.at[0]], bd, sem).wait()

        @plsc.parallel_loop(0, _CH, 1, unroll=4)
        def _(r):
            for q in range(EDGE_C // 16):
                sl = pl.ds(q * 16, 16)
                ob[0, r, sl] = bs[r, sl] + bd[r, sl] + eb[0, r, sl]

        pltpu.sync_copy(ob, out.at[pl.ds(row, 1)])
        return carry

    lax.fori_loop(0, _CPT, chunk, 0)


def _scd_call(srcc, dstc, b_src, b_dst, eeuc):
    f = pl.kernel(
        _scd_body,
        out_type=jax.ShapeDtypeStruct((_NW * _CPT, _CH, EDGE_C), jnp.float32),
        mesh=plsc.VectorSubcoreMesh(**_SC_MESH),
        compiler_params=pltpu.CompilerParams(use_tc_tiling_on_sc=False),
        scratch_types=[
            pltpu.VMEM((1, _CH), jnp.int32),
            pltpu.VMEM((1, _CH), jnp.int32),
            pltpu.VMEM((_CH, EDGE_C), jnp.float32),
            pltpu.VMEM((_CH, EDGE_C), jnp.float32),
            pltpu.VMEM((1, _CH, EDGE_C), jnp.float32),
            pltpu.VMEM((1, _CH, EDGE_C), jnp.float32),
            pltpu.SemaphoreType.DMA,
        ],
    )
    return f(srcc, dstc, b_src, b_dst, eeuc)


_NPT = N // _SC_NS


def _sca_body(srcc, dstc, as16, ad16, eac16, z16,
              exc16, den2,
              sidx, didx, asb, adb, eab, exb, den_sh, sem):
    c = lax.axis_index("c")
    s = lax.axis_index("s")
    wid = s * _SC_NC + c
    row0 = wid * _CPT
    pltpu.sync_copy(z16, den_sh.at[pl.ds(s * _NPT, _NPT)])
    plsc.subcore_barrier()

    def chunk(ch, carry):
        row = row0 + ch
        base = row * _CH
        pltpu.sync_copy(srcc.at[pl.ds(row, 1)], sidx)
        pltpu.sync_copy(dstc.at[pl.ds(row, 1)], didx)
        pltpu.sync_copy(eac16.at[pl.ds(base, _CH)], eab)
        pltpu.async_copy(as16.at[sidx.at[0]], asb, sem).wait()
        pltpu.async_copy(ad16.at[didx.at[0]], adb, sem).wait()

        @plsc.parallel_loop(0, _CH, 1, unroll=4)
        def _(r):
            l = asb[r, :] + adb[r, :] + eab[r, :]
            l = jnp.maximum(l, 0.2 * l)
            exb[r, :] = jnp.exp(l)

        pltpu.sync_copy(exb, exc16.at[pl.ds(base, _CH)])
        pltpu.sync_copy(exb, den_sh.at[didx.at[0]], add=True)
        return carry

    lax.fori_loop(0, _CPT, chunk, 0)
    plsc.subcore_barrier()
    pltpu.sync_copy(den_sh.at[pl.ds(s * _NPT, _NPT)],
                    den2.at[c, pl.ds(s * _NPT, _NPT)])


def _sca_call(srcc, dstc, as16, ad16, eac16, z16):
    f = pl.kernel(
        _sca_body,
        out_type=(
            jax.ShapeDtypeStruct((E, 16), jnp.float32),
            jax.ShapeDtypeStruct((_SC_NC, N, 16), jnp.float32),
        ),
        mesh=plsc.VectorSubcoreMesh(**_SC_MESH),
        compiler_params=pltpu.CompilerParams(use_tc_tiling_on_sc=False),
        scratch_types=[
            pltpu.VMEM((1, _CH), jnp.int32),
            pltpu.VMEM((1, _CH), jnp.int32),
            pltpu.VMEM((_CH, 16), jnp.float32),
            pltpu.VMEM((_CH, 16), jnp.float32),
            pltpu.VMEM((_CH, 16), jnp.float32),
            pltpu.VMEM((_CH, 16), jnp.float32),
            pltpu.VMEM_SHARED((N, 16), jnp.float32),
            pltpu.SemaphoreType.DMA,
        ],
    )
    return f(srcc, dstc, as16, ad16, eac16, z16)


_UC = 48
_NPASS = NC * HEADS * VAL_C // _UC


def _scb_body(srcc, dstc, exc16, dinv16, *rest):
    vps = rest[:_NPASS]
    z = rest[_NPASS]
    u2 = rest[_NPASS + 1]
    sidx, didx, exb, dib, abig, vb, u_sh, sem = rest[_NPASS + 2:]
    c = lax.axis_index("c")
    s = lax.axis_index("s")
    wid = s * _SC_NC + c
    row0 = wid * _CPT

    for p, vp in enumerate(vps):
        pltpu.sync_copy(z, u_sh.at[pl.ds(s * _NPT, _NPT)])
        plsc.subcore_barrier()

        def chunk(ch, carry, p=p, vp=vp):
            row = row0 + ch
            base = row * _CH
            pltpu.sync_copy(srcc.at[pl.ds(row, 1)], sidx)
            pltpu.sync_copy(dstc.at[pl.ds(row, 1)], didx)
            if p == 0:
                pltpu.sync_copy(exc16.at[pl.ds(base, _CH)], exb)
                pltpu.async_copy(dinv16.at[didx.at[0]], dib, sem).wait()

                @plsc.parallel_loop(0, _CH, 1, unroll=4)
                def _(r):
                    abig[ch * _CH + r, :] = exb[r, :] * dib[r, :]

            pltpu.async_copy(vp.at[sidx.at[0]], vb, sem).wait()

            @plsc.parallel_loop(0, _CH, 1, unroll=2)
            def _(r):
                a = abig[ch * _CH + r, :]
                for q in range(_UC // 16):
                    sl = pl.ds(q * 16, 16)
                    vb[r, sl] = vb[r, sl] * a

            pltpu.sync_copy(vb, u_sh.at[didx.at[0]], add=True)
            return carry

        lax.fori_loop(0, _CPT, chunk, 0)
        plsc.subcore_barrier()
        pltpu.sync_copy(u_sh.at[pl.ds(s * _NPT, _NPT)],
                        u2.at[p, c, pl.ds(s * _NPT, _NPT)])
        plsc.subcore_barrier()


def _scb_call(srcc, dstc, exc16, dinv16, vps, z):
    f = pl.kernel(
        _scb_body,
        out_type=jax.ShapeDtypeStruct((_NPASS, _SC_NC, N, _UC), jnp.float32),
        mesh=plsc.VectorSubcoreMesh(**_SC_MESH),
        compiler_params=pltpu.CompilerParams(use_tc_tiling_on_sc=False),
        scratch_types=[
            pltpu.VMEM((1, _CH), jnp.int32),
            pltpu.VMEM((1, _CH), jnp.int32),
            pltpu.VMEM((_CH, 16), jnp.float32),
            pltpu.VMEM((_CH, 16), jnp.float32),
            pltpu.VMEM((_CPT * _CH, 16), jnp.float32),
            pltpu.VMEM((_CH, _UC), jnp.float32),
            pltpu.VMEM_SHARED((N, _UC), jnp.float32),
            pltpu.SemaphoreType.DMA,
        ],
    )
    return f(srcc, dstc, exc16, dinv16, *vps, z)


def kernel(bb_rel, bb_features, edge_features, edge_index, noising_mask,
           W_e1, W_alpha, W_v, W_proj, W_g, W_ff, W_eu1, W_eu2):
    src, dst = edge_index[0], edge_index[1]
    mask_f = noising_mask.astype(jnp.float32)

    x0 = jnp.concatenate(
        [bb_features[:, 0, :], jnp.zeros((N, 2), jnp.float32), mask_f[:, None]],
        axis=-1)
    Wa_s = W_alpha[:TOT_C]
    Wa_d = W_alpha[TOT_C:2 * TOT_C]
    as16 = x0 @ jnp.concatenate([Wa_s, Wa_s], axis=1)
    ad16 = x0 @ jnp.concatenate([Wa_d, Wa_d], axis=1)
    x = jnp.zeros((N, NC, TOT_C), jnp.float32)
    x = x.at[..., :BB_C].set(bb_features)
    x = x.at[:, 1:4, BB_C:].set(jnp.swapaxes(bb_rel, -1, -2))
    x = x.at[:, 0, TOT_C - 1].set(mask_f)
    v_kdh = jnp.einsum('nkc,chd->nkdh', x, W_v).reshape(N, NC * HEADS * VAL_C)
    vps = [v_kdh[:, p * _UC:(p + 1) * _UC] for p in range(_NPASS)]

    Wa_e = W_alpha[2 * TOT_C:]
    Wa_e16 = jnp.concatenate([Wa_e, Wa_e], axis=1)
    Weu_e = W_eu1[2 * BB_C:]
    ea16, e_eu = _edge_pre(edge_features, W_e1, Wa_e16, Weu_e)

    srcc = src.reshape(_NW * _CPT, _CH)
    dstc = dst.reshape(_NW * _CPT, _CH)
    z16 = jnp.zeros((_NPT, 16), jnp.float32)
    exc16, den2 = _sca_call(srcc, dstc, as16, ad16, ea16, z16)
    dinv16 = 1.0 / (den2[0] + den2[1] + 1e-9)

    zuc = jnp.zeros((_NPT, _UC), jnp.float32)
    u2 = _scb_call(srcc, dstc, exc16, dinv16, vps, zuc)
    u = u2[:, 0] + u2[:, 1]
    u = jnp.concatenate(list(u), axis=-1)
    W_proj_dh = (W_proj.reshape(HEADS, VAL_C, BB_C)
                 .transpose(1, 0, 2).reshape(HEADS * VAL_C, BB_C))
    agg = u.reshape(N, NC, HEADS * VAL_C) @ W_proj_dh

    gate = jax.nn.silu(agg[:, 0:1, :] @ W_g)
    new_bb = agg + (agg @ W_ff) * gate

    nb0 = new_bb[:, 0, :]
    b_src = nb0 @ W_eu1[:BB_C]
    b_dst = nb0 @ W_eu1[BB_C:2 * BB_C]
    eeuc = e_eu.reshape(_NW * _CPT, _CH, EDGE_C)
    h_pre = _scd_call(srcc, dstc, b_src, b_dst, eeuc)
    new_edge = _edge_final(h_pre.reshape(E, EDGE_C), W_eu2)
    return new_bb, new_edge


# speedup vs baseline: 11.2380x; 1.0429x over previous
import functools

import jax
import jax.numpy as jnp
from jax import lax
from jax.experimental import pallas as pl
from jax.experimental.pallas import tpu as pltpu
from jax.experimental.pallas import tpu_sc as plsc

N = 10000
E = 160000
NC = 9
BB_C = 32
N_BB = 3
TOT_C = 35
HEADS = 8
VAL_C = 8
EDGE_C = 64

EBLK = 4000


def _edge_pre_kernel(ef_ref, we1_ref, wae_ref, weue_ref, ea_ref, eeu_ref):
    ef = ef_ref[...]
    emb = jax.nn.silu(ef @ we1_ref[...])
    ea_ref[...] = emb @ wae_ref[...]
    eeu_ref[...] = ef @ weue_ref[...]


def _edge_pre(edge_features, W_e1, Wa_e16, Weu_e):
    grid = (E // EBLK,)
    return pl.pallas_call(
        _edge_pre_kernel,
        grid=grid,
        in_specs=[
            pl.BlockSpec((EBLK, EDGE_C), lambda i: (i, 0)),
            pl.BlockSpec((EDGE_C, EDGE_C), lambda i: (0, 0)),
            pl.BlockSpec((EDGE_C, 2 * HEADS), lambda i: (0, 0)),
            pl.BlockSpec((EDGE_C, EDGE_C), lambda i: (0, 0)),
        ],
        out_specs=[
            pl.BlockSpec((EBLK, 2 * HEADS), lambda i: (i, 0)),
            pl.BlockSpec((EBLK, EDGE_C), lambda i: (i, 0)),
        ],
        out_shape=[
            jax.ShapeDtypeStruct((E, 2 * HEADS), jnp.float32),
            jax.ShapeDtypeStruct((E, EDGE_C), jnp.float32),
        ],
    )(edge_features, W_e1, Wa_e16, Weu_e)


def _edge_final_kernel(h_ref, weu2_ref, out_ref):
    h = jax.nn.silu(h_ref[...])
    out_ref[...] = h @ weu2_ref[...]


def _edge_final(h_pre, W_eu2):
    grid = (E // EBLK,)
    return pl.pallas_call(
        _edge_final_kernel,
        grid=grid,
        in_specs=[
            pl.BlockSpec((EBLK, EDGE_C), lambda i: (i, 0)),
            pl.BlockSpec((EDGE_C, EDGE_C), lambda i: (0, 0)),
        ],
        out_specs=pl.BlockSpec((EBLK, EDGE_C), lambda i: (i, 0)),
        out_shape=jax.ShapeDtypeStruct((E, EDGE_C), jnp.float32),
    )(h_pre, W_eu2)


NBLK = 1000


def _node_pre_kernel(bb0_ref, bbf_ref, rel_ref, mask_ref,
                     wv_ref, was_ref, wad_ref,
                     vk_ref, as_ref, ad_ref):
    mask = mask_ref[...]
    zeros2 = jnp.zeros((NBLK, 2), jnp.float32)
    x0 = jnp.concatenate([bb0_ref[...], zeros2, mask], axis=-1)
    as_ref[...] = x0 @ was_ref[...]
    ad_ref[...] = x0 @ wad_ref[...]
    wv = wv_ref[...]
    rel = rel_ref[...]
    for k in range(NC):
        if k == 0:
            xk = x0
        elif 1 <= k <= 3:
            xk = jnp.concatenate(
                [bbf_ref[:, k, :], rel[:, :, k - 1]], axis=-1)
        else:
            xk = jnp.concatenate(
                [bbf_ref[:, k, :], jnp.zeros((NBLK, N_BB), jnp.float32)],
                axis=-1)
        vkk = xk @ wv
        vk_ref[2 * k] = vkk[:, :32]
        vk_ref[2 * k + 1] = vkk[:, 32:]


def _node_pre(bb_features, bb_rel, mask16, Wv_dh, Wa_s16, Wa_d16):
    grid = (N // NBLK,)
    return pl.pallas_call(
        _node_pre_kernel,
        grid=grid,
        in_specs=[
            pl.BlockSpec((NBLK, BB_C), lambda i: (i, 0)),
            pl.BlockSpec((NBLK, NC, BB_C), lambda i: (i, 0, 0)),
            pl.BlockSpec((NBLK, N_BB, 3), lambda i: (i, 0, 0)),
            pl.BlockSpec((NBLK, 1), lambda i: (i, 0)),
            pl.BlockSpec((TOT_C, HEADS * VAL_C), lambda i: (0, 0)),
            pl.BlockSpec((TOT_C, 16), lambda i: (0, 0)),
            pl.BlockSpec((TOT_C, 16), lambda i: (0, 0)),
        ],
        out_specs=[
            pl.BlockSpec((2 * NC, NBLK, 32), lambda i: (0, i, 0)),
            pl.BlockSpec((NBLK, 16), lambda i: (i, 0)),
            pl.BlockSpec((NBLK, 16), lambda i: (i, 0)),
        ],
        out_shape=[
            jax.ShapeDtypeStruct((2 * NC, N, 32), jnp.float32),
            jax.ShapeDtypeStruct((N, 16), jnp.float32),
            jax.ShapeDtypeStruct((N, 16), jnp.float32),
        ],
    )(bb_features[:, 0, :], bb_features, bb_rel, mask16,
      Wv_dh, Wa_s16, Wa_d16)


def _node_post_kernel(u2_ref, wp_ref, wg_ref, wf_ref, ws_ref, wd_ref,
                      nbb_ref, bs_ref, bd_ref):
    wp = wp_ref[...]
    wf = wf_ref[...]

    def uk(k):
        lo = u2_ref[2 * k, 0] + u2_ref[2 * k, 1]
        hi = u2_ref[2 * k + 1, 0] + u2_ref[2 * k + 1, 1]
        return jnp.concatenate([lo, hi], axis=-1)

    agg0 = uk(0) @ wp
    gate = jax.nn.silu(agg0 @ wg_ref[...])
    nb0 = agg0 + (agg0 @ wf) * gate
    nbb_ref[:, 0, :] = nb0
    for k in range(1, NC):
        aggk = uk(k) @ wp
        nbb_ref[:, k, :] = aggk + (aggk @ wf) * gate
    bs_ref[...] = nb0 @ ws_ref[...]
    bd_ref[...] = nb0 @ wd_ref[...]


def _node_post(u2, W_proj_dh, W_g, W_ff, Weu_s, Weu_d):
    grid = (N // NBLK,)
    return pl.pallas_call(
        _node_post_kernel,
        grid=grid,
        in_specs=[
            pl.BlockSpec((2 * NC, _SC_NC, NBLK, 32),
                         lambda i: (0, 0, i, 0)),
            pl.BlockSpec((HEADS * VAL_C, BB_C), lambda i: (0, 0)),
            pl.BlockSpec((BB_C, BB_C), lambda i: (0, 0)),
            pl.BlockSpec((BB_C, BB_C), lambda i: (0, 0)),
            pl.BlockSpec((BB_C, EDGE_C), lambda i: (0, 0)),
            pl.BlockSpec((BB_C, EDGE_C), lambda i: (0, 0)),
        ],
        out_specs=[
            pl.BlockSpec((NBLK, NC, BB_C), lambda i: (i, 0, 0)),
            pl.BlockSpec((NBLK, EDGE_C), lambda i: (i, 0)),
            pl.BlockSpec((NBLK, EDGE_C), lambda i: (i, 0)),
        ],
        out_shape=[
            jax.ShapeDtypeStruct((N, NC, BB_C), jnp.float32),
            jax.ShapeDtypeStruct((N, EDGE_C), jnp.float32),
            jax.ShapeDtypeStruct((N, EDGE_C), jnp.float32),
        ],
    )(u2, W_proj_dh, W_g, W_ff, Weu_s, Weu_d)


_SC_NC = 2
_SC_NS = 16
_NW = _SC_NC * _SC_NS
_CH = 125
_CPT = E // (_NW * _CH)
_SC_MESH = dict(core_axis_name="c", subcore_axis_name="s")


def _scd_body(srcc, dstc, bsrc, bdst, eeuc, out,
              sidx, didx, bs, bd, eb, ob, sem):
    c = lax.axis_index("c")
    s = lax.axis_index("s")
    wid = s * _SC_NC + c
    row0 = wid * _CPT

    def chunk(ch, carry):
        row = row0 + ch
        pltpu.sync_copy(srcc.at[pl.ds(row, 1)], sidx)
        pltpu.sync_copy(dstc.at[pl.ds(row, 1)], didx)
        pltpu.sync_copy(eeuc.at[pl.ds(row, 1)], eb)
        pltpu.async_copy(bsrc.at[sidx.at[0]], bs, sem).wait()
        pltpu.async_copy(bdst.at[didx.at[0]], bd, sem).wait()

        @plsc.parallel_loop(0, _CH, 1, unroll=4)
        def _(r):
            for q in range(EDGE_C // 16):
                sl = pl.ds(q * 16, 16)
                ob[0, r, sl] = bs[r, sl] + bd[r, sl] + eb[0, r, sl]

        pltpu.sync_copy(ob, out.at[pl.ds(row, 1)])
        return carry

    lax.fori_loop(0, _CPT, chunk, 0)


def _scd_call(srcc, dstc, b_src, b_dst, eeuc):
    f = pl.kernel(
        _scd_body,
        out_type=jax.ShapeDtypeStruct((_NW * _CPT, _CH, EDGE_C), jnp.float32),
        mesh=plsc.VectorSubcoreMesh(**_SC_MESH),
        compiler_params=pltpu.CompilerParams(use_tc_tiling_on_sc=False),
        scratch_types=[
            pltpu.VMEM((1, _CH), jnp.int32),
            pltpu.VMEM((1, _CH), jnp.int32),
            pltpu.VMEM((_CH, EDGE_C), jnp.float32),
            pltpu.VMEM((_CH, EDGE_C), jnp.float32),
            pltpu.VMEM((1, _CH, EDGE_C), jnp.float32),
            pltpu.VMEM((1, _CH, EDGE_C), jnp.float32),
            pltpu.SemaphoreType.DMA,
        ],
    )
    return f(srcc, dstc, b_src, b_dst, eeuc)


_NPT = N // _SC_NS


def _sca_body(srcc, dstc, as16, ad16, eac16, z16,
              exc16, den2,
              sidx, didx, asb, adb, eab, exb, den_sh, sem):
    c = lax.axis_index("c")
    s = lax.axis_index("s")
    wid = s * _SC_NC + c
    row0 = wid * _CPT
    pltpu.sync_copy(z16, den_sh.at[pl.ds(s * _NPT, _NPT)])
    plsc.subcore_barrier()

    def chunk(ch, carry):
        row = row0 + ch
        base = row * _CH
        pltpu.sync_copy(srcc.at[pl.ds(row, 1)], sidx)
        pltpu.sync_copy(dstc.at[pl.ds(row, 1)], didx)
        pltpu.sync_copy(eac16.at[pl.ds(base, _CH)], eab)
        pltpu.async_copy(as16.at[sidx.at[0]], asb, sem).wait()
        pltpu.async_copy(ad16.at[didx.at[0]], adb, sem).wait()

        @plsc.parallel_loop(0, _CH, 1, unroll=4)
        def _(r):
            l = asb[r, :] + adb[r, :] + eab[r, :]
            l = jnp.maximum(l, 0.2 * l)
            exb[r, :] = jnp.exp(l)

        pltpu.sync_copy(exb, exc16.at[pl.ds(base, _CH)])
        pltpu.sync_copy(exb, den_sh.at[didx.at[0]], add=True)
        return carry

    lax.fori_loop(0, _CPT, chunk, 0)
    plsc.subcore_barrier()
    pltpu.sync_copy(den_sh.at[pl.ds(s * _NPT, _NPT)],
                    den2.at[c, pl.ds(s * _NPT, _NPT)])


def _sca_call(srcc, dstc, as16, ad16, eac16, z16):
    f = pl.kernel(
        _sca_body,
        out_type=(
            jax.ShapeDtypeStruct((E, 16), jnp.float32),
            jax.ShapeDtypeStruct((_SC_NC, N, 16), jnp.float32),
        ),
        mesh=plsc.VectorSubcoreMesh(**_SC_MESH),
        compiler_params=pltpu.CompilerParams(use_tc_tiling_on_sc=False),
        scratch_types=[
            pltpu.VMEM((1, _CH), jnp.int32),
            pltpu.VMEM((1, _CH), jnp.int32),
            pltpu.VMEM((_CH, 16), jnp.float32),
            pltpu.VMEM((_CH, 16), jnp.float32),
            pltpu.VMEM((_CH, 16), jnp.float32),
            pltpu.VMEM((_CH, 16), jnp.float32),
            pltpu.VMEM_SHARED((N, 16), jnp.float32),
            pltpu.SemaphoreType.DMA,
        ],
    )
    return f(srcc, dstc, as16, ad16, eac16, z16)


_UC = 32
_NPASS = NC * HEADS * VAL_C // _UC


def _scb_body(srcc, dstc, exc16, dinv16, vk, z,
              u2,
              sidx, didx, exb, dib, abig, vb, u_sh, gsem0, gsem1):
    c = lax.axis_index("c")
    s = lax.axis_index("s")
    wid = s * _SC_NC + c
    row0 = wid * _CPT
    gsems = (gsem0, gsem1)

    def alpha_chunk(ch, carry):
        row = row0 + ch
        base = row * _CH
        pltpu.sync_copy(dstc.at[pl.ds(row, 1)], sidx.at[pl.ds(0, 1)])
        pltpu.sync_copy(exc16.at[pl.ds(base, _CH)], exb)
        pltpu.async_copy(dinv16.at[sidx.at[0]], dib, gsem0).wait()

        @plsc.parallel_loop(0, _CH, 1, unroll=4)
        def _(r):
            abig[ch * _CH + r, :] = exb[r, :] * dib[r, :]

        return carry

    lax.fori_loop(0, _CPT, alpha_chunk, 0)

    for p in range(_NPASS):
        vp = vk.at[p]
        pltpu.sync_copy(z, u_sh.at[pl.ds(s * _NPT, _NPT)])
        plsc.subcore_barrier()

        for b in range(2):
            pltpu.sync_copy(srcc.at[pl.ds(row0 + b, 1)],
                            sidx.at[pl.ds(b, 1)])
            pltpu.sync_copy(dstc.at[pl.ds(row0 + b, 1)],
                            didx.at[pl.ds(b, 1)])
            pltpu.async_copy(vp.at[sidx.at[b]], vb.at[b], gsems[b])

        def pair(g, carry, vp=vp):
            for b in range(2):
                ch = g * 2 + b
                pltpu.make_async_copy(vp.at[sidx.at[b]], vb.at[b],
                                      gsems[b]).wait()

                @plsc.parallel_loop(0, _CH, 1, unroll=2)
                def _(r):
                    a = abig[ch * _CH + r, :]
                    for q in range(_UC // 16):
                        sl = pl.ds(q * 16, 16)
                        vb[b, r, sl] = vb[b, r, sl] * a

                pltpu.sync_copy(vb.at[b], u_sh.at[didx.at[b]], add=True)

                @pl.when(ch + 2 < _CPT)
                def _(b=b, ch=ch):
                    row = row0 + ch + 2
                    pltpu.sync_copy(srcc.at[pl.ds(row, 1)],
                                    sidx.at[pl.ds(b, 1)])
                    pltpu.sync_copy(dstc.at[pl.ds(row, 1)],
                                    didx.at[pl.ds(b, 1)])
                    pltpu.async_copy(vp.at[sidx.at[b]], vb.at[b], gsems[b])

            return carry

        lax.fori_loop(0, _CPT // 2, pair, 0)
        plsc.subcore_barrier()
        pltpu.sync_copy(u_sh.at[pl.ds(s * _NPT, _NPT)],
                        u2.at[p, c, pl.ds(s * _NPT, _NPT)])
        plsc.subcore_barrier()


def _scb_call(srcc, dstc, exc16, dinv16, vk, z):
    f = pl.kernel(
        _scb_body,
        out_type=jax.ShapeDtypeStruct((_NPASS, _SC_NC, N, _UC), jnp.float32),
        mesh=plsc.VectorSubcoreMesh(**_SC_MESH),
        compiler_params=pltpu.CompilerParams(use_tc_tiling_on_sc=False),
        scratch_types=[
            pltpu.VMEM((2, _CH), jnp.int32),
            pltpu.VMEM((2, _CH), jnp.int32),
            pltpu.VMEM((_CH, 16), jnp.float32),
            pltpu.VMEM((_CH, 16), jnp.float32),
            pltpu.VMEM((_CPT * _CH, 16), jnp.float32),
            pltpu.VMEM((2, _CH, _UC), jnp.float32),
            pltpu.VMEM_SHARED((N, _UC), jnp.float32),
            pltpu.SemaphoreType.DMA,
            pltpu.SemaphoreType.DMA,
        ],
    )
    return f(srcc, dstc, exc16, dinv16, vk, z)


def kernel(bb_rel, bb_features, edge_features, edge_index, noising_mask,
           W_e1, W_alpha, W_v, W_proj, W_g, W_ff, W_eu1, W_eu2):
    src, dst = edge_index[0], edge_index[1]
    mask_f = noising_mask.astype(jnp.float32)

    Wa_s = W_alpha[:TOT_C]
    Wa_d = W_alpha[TOT_C:2 * TOT_C]
    Wa_s16 = jnp.concatenate([Wa_s, Wa_s], axis=1)
    Wa_d16 = jnp.concatenate([Wa_d, Wa_d], axis=1)
    Wv_dh = jnp.swapaxes(W_v, 1, 2).reshape(TOT_C, HEADS * VAL_C)
    vk, as16, ad16 = _node_pre(bb_features, bb_rel, mask_f[:, None],
                               Wv_dh, Wa_s16, Wa_d16)

    Wa_e = W_alpha[2 * TOT_C:]
    Wa_e16 = jnp.concatenate([Wa_e, Wa_e], axis=1)
    Weu_e = W_eu1[2 * BB_C:]
    ea16, e_eu = _edge_pre(edge_features, W_e1, Wa_e16, Weu_e)

    srcc = src.reshape(_NW * _CPT, _CH)
    dstc = dst.reshape(_NW * _CPT, _CH)
    z16 = jnp.zeros((_NPT, 16), jnp.float32)
    exc16, den2 = _sca_call(srcc, dstc, as16, ad16, ea16, z16)
    dinv16 = 1.0 / (den2[0] + den2[1] + 1e-9)

    zuc = jnp.zeros((_NPT, _UC), jnp.float32)
    u2 = _scb_call(srcc, dstc, exc16, dinv16, vk, zuc)
    W_proj_dh = (W_proj.reshape(HEADS, VAL_C, BB_C)
                 .transpose(1, 0, 2).reshape(HEADS * VAL_C, BB_C))

    new_bb, b_src, b_dst = _node_post(u2, W_proj_dh, W_g, W_ff,
                                      W_eu1[:BB_C], W_eu1[BB_C:2 * BB_C])

    eeuc = e_eu.reshape(_NW * _CPT, _CH, EDGE_C)
    h_pre = _scd_call(srcc, dstc, b_src, b_dst, eeuc)
    new_edge = _edge_final(h_pre.reshape(E, EDGE_C), W_eu2)
    return new_bb, new_edge


# speedup vs baseline: 14.9383x; 1.3293x over previous
import functools

import jax
import jax.numpy as jnp
from jax import lax
from jax.experimental import pallas as pl
from jax.experimental.pallas import tpu as pltpu
from jax.experimental.pallas import tpu_sc as plsc

N = 10000
E = 160000
NC = 9
BB_C = 32
N_BB = 3
TOT_C = 35
HEADS = 8
VAL_C = 8
EDGE_C = 64

EBLK = 4000


def _edge_pre_kernel(ef_ref, we1_ref, wae_ref, weue_ref, ea_ref, eeu_ref):
    ef = ef_ref[...]
    emb = jax.nn.silu(ef @ we1_ref[...])
    ea_ref[...] = emb @ wae_ref[...]
    eeu_ref[...] = ef @ weue_ref[...]


def _edge_pre(edge_features, W_e1, Wa_e16, Weu_e):
    grid = (E // EBLK,)
    return pl.pallas_call(
        _edge_pre_kernel,
        grid=grid,
        in_specs=[
            pl.BlockSpec((EBLK, EDGE_C), lambda i: (i, 0)),
            pl.BlockSpec((EDGE_C, EDGE_C), lambda i: (0, 0)),
            pl.BlockSpec((EDGE_C, 2 * HEADS), lambda i: (0, 0)),
            pl.BlockSpec((EDGE_C, EDGE_C), lambda i: (0, 0)),
        ],
        out_specs=[
            pl.BlockSpec((EBLK, 2 * HEADS), lambda i: (i, 0)),
            pl.BlockSpec((EBLK, EDGE_C), lambda i: (i, 0)),
        ],
        out_shape=[
            jax.ShapeDtypeStruct((E, 2 * HEADS), jnp.float32),
            jax.ShapeDtypeStruct((E, EDGE_C), jnp.float32),
        ],
    )(edge_features, W_e1, Wa_e16, Weu_e)


def _edge_final_kernel(h_ref, weu2_ref, out_ref):
    h = jax.nn.silu(h_ref[...])
    out_ref[...] = h @ weu2_ref[...]


def _edge_final(h_pre, W_eu2):
    grid = (E // EBLK,)
    return pl.pallas_call(
        _edge_final_kernel,
        grid=grid,
        in_specs=[
            pl.BlockSpec((EBLK, EDGE_C), lambda i: (i, 0)),
            pl.BlockSpec((EDGE_C, EDGE_C), lambda i: (0, 0)),
        ],
        out_specs=pl.BlockSpec((EBLK, EDGE_C), lambda i: (i, 0)),
        out_shape=jax.ShapeDtypeStruct((E, EDGE_C), jnp.float32),
    )(h_pre, W_eu2)


NBLK = 1000


def _node_pre_kernel(bb0_ref, bbf_ref, rel_ref, mask_ref,
                     wv_ref, was_ref, wad_ref,
                     vk_ref, as_ref, ad_ref):
    mask = mask_ref[...]
    zeros2 = jnp.zeros((NBLK, 2), jnp.float32)
    x0 = jnp.concatenate([bb0_ref[...], zeros2, mask], axis=-1)
    as_ref[...] = x0 @ was_ref[...]
    ad_ref[...] = x0 @ wad_ref[...]
    wv = wv_ref[...]
    rel = rel_ref[...]
    vs = []
    for k in range(NC):
        if k == 0:
            xk = x0
        elif 1 <= k <= 3:
            xk = jnp.concatenate(
                [bbf_ref[:, k, :], rel[:, :, k - 1]], axis=-1)
        else:
            xk = jnp.concatenate(
                [bbf_ref[:, k, :], jnp.zeros((NBLK, N_BB), jnp.float32)],
                axis=-1)
        vs.append(xk @ wv)
    vfull = jnp.concatenate(vs, axis=-1)
    for p in range(_NPASS):
        vk_ref[p] = vfull[:, p * _UC:(p + 1) * _UC]


def _node_pre(bb_features, bb_rel, mask16, Wv_dh, Wa_s16, Wa_d16):
    grid = (N // NBLK,)
    return pl.pallas_call(
        _node_pre_kernel,
        grid=grid,
        in_specs=[
            pl.BlockSpec((NBLK, BB_C), lambda i: (i, 0)),
            pl.BlockSpec((NBLK, NC, BB_C), lambda i: (i, 0, 0)),
            pl.BlockSpec((NBLK, N_BB, 3), lambda i: (i, 0, 0)),
            pl.BlockSpec((NBLK, 1), lambda i: (i, 0)),
            pl.BlockSpec((TOT_C, HEADS * VAL_C), lambda i: (0, 0)),
            pl.BlockSpec((TOT_C, 16), lambda i: (0, 0)),
            pl.BlockSpec((TOT_C, 16), lambda i: (0, 0)),
        ],
        out_specs=[
            pl.BlockSpec((_NPASS, NBLK, _UC), lambda i: (0, i, 0)),
            pl.BlockSpec((NBLK, 16), lambda i: (i, 0)),
            pl.BlockSpec((NBLK, 16), lambda i: (i, 0)),
        ],
        out_shape=[
            jax.ShapeDtypeStruct((_NPASS, N, _UC), jnp.float32),
            jax.ShapeDtypeStruct((N, 16), jnp.float32),
            jax.ShapeDtypeStruct((N, 16), jnp.float32),
        ],
    )(bb_features[:, 0, :], bb_features, bb_rel, mask16,
      Wv_dh, Wa_s16, Wa_d16)


def _node_post_kernel(u2_ref, wp_ref, wg_ref, wf_ref, ws_ref, wd_ref,
                      nbb_ref, bs_ref, bd_ref):
    wp = wp_ref[...]
    wf = wf_ref[...]
    ufull = jnp.concatenate(
        [u2_ref[p, 0] + u2_ref[p, 1] for p in range(_NPASS)],
        axis=-1)

    def uk(k):
        return ufull[:, k * 64:(k + 1) * 64]

    agg0 = uk(0) @ wp
    gate = jax.nn.silu(agg0 @ wg_ref[...])
    nb0 = agg0 + (agg0 @ wf) * gate
    nbb_ref[:, 0, :] = nb0
    for k in range(1, NC):
        aggk = uk(k) @ wp
        nbb_ref[:, k, :] = aggk + (aggk @ wf) * gate
    bs_ref[...] = nb0 @ ws_ref[...]
    bd_ref[...] = nb0 @ wd_ref[...]


def _node_post(u2, W_proj_dh, W_g, W_ff, Weu_s, Weu_d):
    grid = (N // NBLK,)
    return pl.pallas_call(
        _node_post_kernel,
        grid=grid,
        in_specs=[
            pl.BlockSpec((_NPASS, _SC_NC, NBLK, _UC),
                         lambda i: (0, 0, i, 0)),
            pl.BlockSpec((HEADS * VAL_C, BB_C), lambda i: (0, 0)),
            pl.BlockSpec((BB_C, BB_C), lambda i: (0, 0)),
            pl.BlockSpec((BB_C, BB_C), lambda i: (0, 0)),
            pl.BlockSpec((BB_C, EDGE_C), lambda i: (0, 0)),
            pl.BlockSpec((BB_C, EDGE_C), lambda i: (0, 0)),
        ],
        out_specs=[
            pl.BlockSpec((NBLK, NC, BB_C), lambda i: (i, 0, 0)),
            pl.BlockSpec((NBLK, EDGE_C), lambda i: (i, 0)),
            pl.BlockSpec((NBLK, EDGE_C), lambda i: (i, 0)),
        ],
        out_shape=[
            jax.ShapeDtypeStruct((N, NC, BB_C), jnp.float32),
            jax.ShapeDtypeStruct((N, EDGE_C), jnp.float32),
            jax.ShapeDtypeStruct((N, EDGE_C), jnp.float32),
        ],
    )(u2, W_proj_dh, W_g, W_ff, Weu_s, Weu_d)


_SC_NC = 2
_SC_NS = 16
_NW = _SC_NC * _SC_NS
_CH = 125
_CPT = E // (_NW * _CH)
_SC_MESH = dict(core_axis_name="c", subcore_axis_name="s")


def _scd_body(srcc, dstc, bsrc, bdst, eeuc, out,
              sidx, didx, bs, bd, eb, ob, sem):
    c = lax.axis_index("c")
    s = lax.axis_index("s")
    wid = s * _SC_NC + c
    row0 = wid * _CPT

    def chunk(ch, carry):
        row = row0 + ch
        pltpu.sync_copy(srcc.at[pl.ds(row, 1)], sidx)
        pltpu.sync_copy(dstc.at[pl.ds(row, 1)], didx)
        pltpu.sync_copy(eeuc.at[pl.ds(row, 1)], eb)
        pltpu.async_copy(bsrc.at[sidx.at[0]], bs, sem).wait()
        pltpu.async_copy(bdst.at[didx.at[0]], bd, sem).wait()

        @plsc.parallel_loop(0, _CH, 1, unroll=4)
        def _(r):
            for q in range(EDGE_C // 16):
                sl = pl.ds(q * 16, 16)
                ob[0, r, sl] = bs[r, sl] + bd[r, sl] + eb[0, r, sl]

        pltpu.sync_copy(ob, out.at[pl.ds(row, 1)])
        return carry

    lax.fori_loop(0, _CPT, chunk, 0)


def _scd_call(srcc, dstc, b_src, b_dst, eeuc):
    f = pl.kernel(
        _scd_body,
        out_type=jax.ShapeDtypeStruct((_NW * _CPT, _CH, EDGE_C), jnp.float32),
        mesh=plsc.VectorSubcoreMesh(**_SC_MESH),
        compiler_params=pltpu.CompilerParams(use_tc_tiling_on_sc=False),
        scratch_types=[
            pltpu.VMEM((1, _CH), jnp.int32),
            pltpu.VMEM((1, _CH), jnp.int32),
            pltpu.VMEM((_CH, EDGE_C), jnp.float32),
            pltpu.VMEM((_CH, EDGE_C), jnp.float32),
            pltpu.VMEM((1, _CH, EDGE_C), jnp.float32),
            pltpu.VMEM((1, _CH, EDGE_C), jnp.float32),
            pltpu.SemaphoreType.DMA,
        ],
    )
    return f(srcc, dstc, b_src, b_dst, eeuc)


_NPT = N // _SC_NS


def _sca_body(srcc, dstc, as16, ad16, eac16, z16,
              exc16, den2,
              sidx, didx, asb, adb, eab, exb, den_sh, sem):
    c = lax.axis_index("c")
    s = lax.axis_index("s")
    wid = s * _SC_NC + c
    row0 = wid * _CPT
    pltpu.sync_copy(z16, den_sh.at[pl.ds(s * _NPT, _NPT)])
    plsc.subcore_barrier()

    def chunk(ch, carry):
        row = row0 + ch
        base = row * _CH
        pltpu.sync_copy(srcc.at[pl.ds(row, 1)], sidx)
        pltpu.sync_copy(dstc.at[pl.ds(row, 1)], didx)
        pltpu.sync_copy(eac16.at[pl.ds(base, _CH)], eab)
        pltpu.async_copy(as16.at[sidx.at[0]], asb, sem).wait()
        pltpu.async_copy(ad16.at[didx.at[0]], adb, sem).wait()

        @plsc.parallel_loop(0, _CH, 1, unroll=4)
        def _(r):
            l = asb[r, :] + adb[r, :] + eab[r, :]
            l = jnp.maximum(l, 0.2 * l)
            exb[r, :] = jnp.exp(l)

        pltpu.sync_copy(exb, exc16.at[pl.ds(base, _CH)])
        pltpu.sync_copy(exb, den_sh.at[didx.at[0]], add=True)
        return carry

    lax.fori_loop(0, _CPT, chunk, 0)
    plsc.subcore_barrier()
    pltpu.sync_copy(den_sh.at[pl.ds(s * _NPT, _NPT)],
                    den2.at[c, pl.ds(s * _NPT, _NPT)])


def _sca_call(srcc, dstc, as16, ad16, eac16, z16):
    f = pl.kernel(
        _sca_body,
        out_type=(
            jax.ShapeDtypeStruct((E, 16), jnp.float32),
            jax.ShapeDtypeStruct((_SC_NC, N, 16), jnp.float32),
        ),
        mesh=plsc.VectorSubcoreMesh(**_SC_MESH),
        compiler_params=pltpu.CompilerParams(use_tc_tiling_on_sc=False),
        scratch_types=[
            pltpu.VMEM((1, _CH), jnp.int32),
            pltpu.VMEM((1, _CH), jnp.int32),
            pltpu.VMEM((_CH, 16), jnp.float32),
            pltpu.VMEM((_CH, 16), jnp.float32),
            pltpu.VMEM((_CH, 16), jnp.float32),
            pltpu.VMEM((_CH, 16), jnp.float32),
            pltpu.VMEM_SHARED((N, 16), jnp.float32),
            pltpu.SemaphoreType.DMA,
        ],
    )
    return f(srcc, dstc, as16, ad16, eac16, z16)


_UC = 32
_NPASS = NC * HEADS * VAL_C // _UC


def _scb_body(srcc, dstc, exc16, dinv16, vk, z,
              u2,
              smeta, dmeta, dib, abig, vb, u_sh, gsem0, gsem1):
    c = lax.axis_index("c")
    s = lax.axis_index("s")
    wid = s * _SC_NC + c
    row0 = wid * _CPT
    gsems = (gsem0, gsem1)

    pltpu.sync_copy(srcc.at[pl.ds(row0, _CPT)], smeta)
    pltpu.sync_copy(dstc.at[pl.ds(row0, _CPT)], dmeta)
    pltpu.sync_copy(exc16.at[pl.ds(row0 * _CH, _CPT * _CH)], abig)

    def alpha_chunk(ch, carry):
        pltpu.async_copy(dinv16.at[dmeta.at[ch]], dib, gsem0).wait()

        @plsc.parallel_loop(0, _CH, 1, unroll=4)
        def _(r):
            abig[ch * _CH + r, :] = abig[ch * _CH + r, :] * dib[r, :]

        return carry

    lax.fori_loop(0, _CPT, alpha_chunk, 0)

    for p in range(_NPASS):
        vp = vk.at[p]
        pltpu.sync_copy(z, u_sh.at[pl.ds(s * _NPT, _NPT)])
        plsc.subcore_barrier()

        for b in range(2):
            pltpu.async_copy(vp.at[smeta.at[b]], vb.at[b], gsems[b])

        def pair(g, carry, vp=vp):
            for b in range(2):
                ch = g * 2 + b
                pltpu.make_async_copy(vp.at[smeta.at[0]], vb.at[b],
                                      gsems[b]).wait()

                @plsc.parallel_loop(0, _CH, 1, unroll=4)
                def _(r):
                    a = abig[ch * _CH + r, :]
                    for q in range(_UC // 16):
                        sl = pl.ds(q * 16, 16)
                        vb[b, r, sl] = vb[b, r, sl] * a

                pltpu.sync_copy(vb.at[b], u_sh.at[dmeta.at[ch]], add=True)

                @pl.when(ch + 2 < _CPT)
                def _(b=b, ch=ch):
                    pltpu.async_copy(vp.at[smeta.at[ch + 2]], vb.at[b],
                                     gsems[b])

            return carry

        lax.fori_loop(0, _CPT // 2, pair, 0)
        plsc.subcore_barrier()
        pltpu.sync_copy(u_sh.at[pl.ds(s * _NPT, _NPT)],
                        u2.at[p, c, pl.ds(s * _NPT, _NPT)])
        plsc.subcore_barrier()


def _scb_call(srcc, dstc, exc16, dinv16, vk, z):
    f = pl.kernel(
        _scb_body,
        out_type=jax.ShapeDtypeStruct((_NPASS, _SC_NC, N, _UC), jnp.float32),
        mesh=plsc.VectorSubcoreMesh(**_SC_MESH),
        compiler_params=pltpu.CompilerParams(use_tc_tiling_on_sc=False),
        scratch_types=[
            pltpu.VMEM((_CPT, _CH), jnp.int32),
            pltpu.VMEM((_CPT, _CH), jnp.int32),
            pltpu.VMEM((_CH, 16), jnp.float32),
            pltpu.VMEM((_CPT * _CH, 16), jnp.float32),
            pltpu.VMEM((2, _CH, _UC), jnp.float32),
            pltpu.VMEM_SHARED((N, _UC), jnp.float32),
            pltpu.SemaphoreType.DMA,
            pltpu.SemaphoreType.DMA,
        ],
    )
    return f(srcc, dstc, exc16, dinv16, vk, z)


def kernel(bb_rel, bb_features, edge_features, edge_index, noising_mask,
           W_e1, W_alpha, W_v, W_proj, W_g, W_ff, W_eu1, W_eu2):
    src, dst = edge_index[0], edge_index[1]
    mask_f = noising_mask.astype(jnp.float32)

    Wa_s = W_alpha[:TOT_C]
    Wa_d = W_alpha[TOT_C:2 * TOT_C]
    Wa_s16 = jnp.concatenate([Wa_s, Wa_s], axis=1)
    Wa_d16 = jnp.concatenate([Wa_d, Wa_d], axis=1)
    Wv_dh = jnp.swapaxes(W_v, 1, 2).reshape(TOT_C, HEADS * VAL_C)
    vk, as16, ad16 = _node_pre(bb_features, bb_rel, mask_f[:, None],
                               Wv_dh, Wa_s16, Wa_d16)

    Wa_e = W_alpha[2 * TOT_C:]
    Wa_e16 = jnp.concatenate([Wa_e, Wa_e], axis=1)
    Weu_e = W_eu1[2 * BB_C:]
    ea16, e_eu = _edge_pre(edge_features, W_e1, Wa_e16, Weu_e)

    srcc = src.reshape(_NW * _CPT, _CH)
    dstc = dst.reshape(_NW * _CPT, _CH)
    z16 = jnp.zeros((_NPT, 16), jnp.float32)
    exc16, den2 = _sca_call(srcc, dstc, as16, ad16, ea16, z16)
    dinv16 = 1.0 / (den2[0] + den2[1] + 1e-9)

    zuc = jnp.zeros((_NPT, _UC), jnp.float32)
    u2 = _scb_call(srcc, dstc, exc16, dinv16, vk, zuc)
    W_proj_dh = (W_proj.reshape(HEADS, VAL_C, BB_C)
                 .transpose(1, 0, 2).reshape(HEADS * VAL_C, BB_C))

    new_bb, b_src, b_dst = _node_post(u2, W_proj_dh, W_g, W_ff,
                                      W_eu1[:BB_C], W_eu1[BB_C:2 * BB_C])

    eeuc = e_eu.reshape(_NW * _CPT, _CH, EDGE_C)
    h_pre = _scd_call(srcc, dstc, b_src, b_dst, eeuc)
    new_edge = _edge_final(h_pre.reshape(E, EDGE_C), W_eu2)
    return new_bb, new_edge


# speedup vs baseline: 17.6641x; 1.1825x over previous
import functools

import jax
import jax.numpy as jnp
from jax import lax
from jax.experimental import pallas as pl
from jax.experimental.pallas import tpu as pltpu
from jax.experimental.pallas import tpu_sc as plsc

N = 10000
E = 160000
NC = 9
BB_C = 32
N_BB = 3
TOT_C = 35
HEADS = 8
VAL_C = 8
EDGE_C = 64

EBLK = 4000


def _edge_pre_kernel(ef_ref, we1_ref, wae_ref, weue_ref, ea_ref, eeu_ref):
    ef = ef_ref[...]
    emb = jax.nn.silu(ef @ we1_ref[...])
    ea_ref[...] = emb @ wae_ref[...]
    eeu_ref[...] = ef @ weue_ref[...]


def _edge_pre(edge_features, W_e1, Wa_e16, Weu_e):
    grid = (E // EBLK,)
    return pl.pallas_call(
        _edge_pre_kernel,
        grid=grid,
        in_specs=[
            pl.BlockSpec((EBLK, EDGE_C), lambda i: (i, 0)),
            pl.BlockSpec((EDGE_C, EDGE_C), lambda i: (0, 0)),
            pl.BlockSpec((EDGE_C, 2 * HEADS), lambda i: (0, 0)),
            pl.BlockSpec((EDGE_C, EDGE_C), lambda i: (0, 0)),
        ],
        out_specs=[
            pl.BlockSpec((EBLK, 2 * HEADS), lambda i: (i, 0)),
            pl.BlockSpec((EBLK, EDGE_C), lambda i: (i, 0)),
        ],
        out_shape=[
            jax.ShapeDtypeStruct((E, 2 * HEADS), jnp.float32),
            jax.ShapeDtypeStruct((E, EDGE_C), jnp.float32),
        ],
    )(edge_features, W_e1, Wa_e16, Weu_e)


def _edge_final_kernel(h_ref, weu2_ref, out_ref):
    h = jax.nn.silu(h_ref[...])
    out_ref[...] = h @ weu2_ref[...]


def _edge_final(h_pre, W_eu2):
    grid = (E // EBLK,)
    return pl.pallas_call(
        _edge_final_kernel,
        grid=grid,
        in_specs=[
            pl.BlockSpec((EBLK, EDGE_C), lambda i: (i, 0)),
            pl.BlockSpec((EDGE_C, EDGE_C), lambda i: (0, 0)),
        ],
        out_specs=pl.BlockSpec((EBLK, EDGE_C), lambda i: (i, 0)),
        out_shape=jax.ShapeDtypeStruct((E, EDGE_C), jnp.float32),
    )(h_pre, W_eu2)


NBLK = 1000


def _node_pre_kernel(bb0_ref, bbf_ref, rel_ref, mask_ref,
                     wv_ref, was_ref, wad_ref,
                     vk_ref, as_ref, ad_ref):
    mask = mask_ref[...]
    zeros2 = jnp.zeros((NBLK, 2), jnp.float32)
    x0 = jnp.concatenate([bb0_ref[...], zeros2, mask], axis=-1)
    as_ref[...] = x0 @ was_ref[...]
    ad_ref[...] = x0 @ wad_ref[...]
    wv = wv_ref[...]
    rel = rel_ref[...]
    vs = []
    for k in range(NC):
        if k == 0:
            xk = x0
        elif 1 <= k <= 3:
            xk = jnp.concatenate(
                [bbf_ref[:, k, :], rel[:, :, k - 1]], axis=-1)
        else:
            xk = jnp.concatenate(
                [bbf_ref[:, k, :], jnp.zeros((NBLK, N_BB), jnp.float32)],
                axis=-1)
        vs.append(xk @ wv)
    vfull = jnp.concatenate(vs, axis=-1)
    for p in range(_NPASS):
        vk_ref[p] = vfull[:, p * _UC:(p + 1) * _UC]


def _node_pre(bb_features, bb_rel, mask16, Wv_dh, Wa_s16, Wa_d16):
    grid = (N // NBLK,)
    return pl.pallas_call(
        _node_pre_kernel,
        grid=grid,
        in_specs=[
            pl.BlockSpec((NBLK, BB_C), lambda i: (i, 0)),
            pl.BlockSpec((NBLK, NC, BB_C), lambda i: (i, 0, 0)),
            pl.BlockSpec((NBLK, N_BB, 3), lambda i: (i, 0, 0)),
            pl.BlockSpec((NBLK, 1), lambda i: (i, 0)),
            pl.BlockSpec((TOT_C, HEADS * VAL_C), lambda i: (0, 0)),
            pl.BlockSpec((TOT_C, 16), lambda i: (0, 0)),
            pl.BlockSpec((TOT_C, 16), lambda i: (0, 0)),
        ],
        out_specs=[
            pl.BlockSpec((_NPASS, NBLK, _UC), lambda i: (0, i, 0)),
            pl.BlockSpec((NBLK, 16), lambda i: (i, 0)),
            pl.BlockSpec((NBLK, 16), lambda i: (i, 0)),
        ],
        out_shape=[
            jax.ShapeDtypeStruct((_NPASS, N, _UC), jnp.float32),
            jax.ShapeDtypeStruct((N, 16), jnp.float32),
            jax.ShapeDtypeStruct((N, 16), jnp.float32),
        ],
    )(bb_features[:, 0, :], bb_features, bb_rel, mask16,
      Wv_dh, Wa_s16, Wa_d16)


def _node_post_kernel(u2_ref, wp_ref, wg_ref, wf_ref, ws_ref, wd_ref,
                      nbb_ref, bs_ref, bd_ref):
    wp = wp_ref[...]
    wf = wf_ref[...]
    ufull = jnp.concatenate(
        [u2_ref[p, 0] + u2_ref[p, 1] for p in range(_NPASS)],
        axis=-1)

    def uk(k):
        return ufull[:, k * 64:(k + 1) * 64]

    agg0 = uk(0) @ wp
    gate = jax.nn.silu(agg0 @ wg_ref[...])
    nb0 = agg0 + (agg0 @ wf) * gate
    nbb_ref[:, 0, :] = nb0
    for k in range(1, NC):
        aggk = uk(k) @ wp
        nbb_ref[:, k, :] = aggk + (aggk @ wf) * gate
    bs_ref[...] = nb0 @ ws_ref[...]
    bd_ref[...] = nb0 @ wd_ref[...]


def _node_post(u2, W_proj_dh, W_g, W_ff, Weu_s, Weu_d):
    grid = (N // NBLK,)
    return pl.pallas_call(
        _node_post_kernel,
        grid=grid,
        in_specs=[
            pl.BlockSpec((_NPASS, _SC_NC, NBLK, _UC),
                         lambda i: (0, 0, i, 0)),
            pl.BlockSpec((HEADS * VAL_C, BB_C), lambda i: (0, 0)),
            pl.BlockSpec((BB_C, BB_C), lambda i: (0, 0)),
            pl.BlockSpec((BB_C, BB_C), lambda i: (0, 0)),
            pl.BlockSpec((BB_C, EDGE_C), lambda i: (0, 0)),
            pl.BlockSpec((BB_C, EDGE_C), lambda i: (0, 0)),
        ],
        out_specs=[
            pl.BlockSpec((NBLK, NC, BB_C), lambda i: (i, 0, 0)),
            pl.BlockSpec((NBLK, EDGE_C), lambda i: (i, 0)),
            pl.BlockSpec((NBLK, EDGE_C), lambda i: (i, 0)),
        ],
        out_shape=[
            jax.ShapeDtypeStruct((N, NC, BB_C), jnp.float32),
            jax.ShapeDtypeStruct((N, EDGE_C), jnp.float32),
            jax.ShapeDtypeStruct((N, EDGE_C), jnp.float32),
        ],
    )(u2, W_proj_dh, W_g, W_ff, Weu_s, Weu_d)


_SC_NC = 2
_SC_NS = 16
_NW = _SC_NC * _SC_NS
_CH = 125
_CPT = E // (_NW * _CH)
_SC_MESH = dict(core_axis_name="c", subcore_axis_name="s")


def _scd_body(srcc, dstc, bsrc, bdst, eeuc, out,
              sidx, didx, bs, bd, eb, ob, sem):
    c = lax.axis_index("c")
    s = lax.axis_index("s")
    wid = s * _SC_NC + c
    row0 = wid * _CPT

    def chunk(ch, carry):
        row = row0 + ch
        pltpu.sync_copy(srcc.at[pl.ds(row, 1)], sidx)
        pltpu.sync_copy(dstc.at[pl.ds(row, 1)], didx)
        pltpu.sync_copy(eeuc.at[pl.ds(row, 1)], eb)
        pltpu.async_copy(bsrc.at[sidx.at[0]], bs, sem).wait()
        pltpu.async_copy(bdst.at[didx.at[0]], bd, sem).wait()

        @plsc.parallel_loop(0, _CH, 1, unroll=4)
        def _(r):
            for q in range(EDGE_C // 16):
                sl = pl.ds(q * 16, 16)
                ob[0, r, sl] = bs[r, sl] + bd[r, sl] + eb[0, r, sl]

        pltpu.sync_copy(ob, out.at[pl.ds(row, 1)])
        return carry

    lax.fori_loop(0, _CPT, chunk, 0)


def _scd_call(srcc, dstc, b_src, b_dst, eeuc):
    f = pl.kernel(
        _scd_body,
        out_type=jax.ShapeDtypeStruct((_NW * _CPT, _CH, EDGE_C), jnp.float32),
        mesh=plsc.VectorSubcoreMesh(**_SC_MESH),
        compiler_params=pltpu.CompilerParams(use_tc_tiling_on_sc=False),
        scratch_types=[
            pltpu.VMEM((1, _CH), jnp.int32),
            pltpu.VMEM((1, _CH), jnp.int32),
            pltpu.VMEM((_CH, EDGE_C), jnp.float32),
            pltpu.VMEM((_CH, EDGE_C), jnp.float32),
            pltpu.VMEM((1, _CH, EDGE_C), jnp.float32),
            pltpu.VMEM((1, _CH, EDGE_C), jnp.float32),
            pltpu.SemaphoreType.DMA,
        ],
    )
    return f(srcc, dstc, b_src, b_dst, eeuc)


_NPT = N // _SC_NS


def _sca_body(srcc, dstc, as16, ad16, eac16, z16,
              exc16, den2,
              sidx, didx, asb, adb, eab, exb, den_sh, sem):
    c = lax.axis_index("c")
    s = lax.axis_index("s")
    wid = s * _SC_NC + c
    row0 = wid * _CPT
    pltpu.sync_copy(z16, den_sh.at[pl.ds(s * _NPT, _NPT)])
    plsc.subcore_barrier()

    def chunk(ch, carry):
        row = row0 + ch
        base = row * _CH
        pltpu.sync_copy(srcc.at[pl.ds(row, 1)], sidx)
        pltpu.sync_copy(dstc.at[pl.ds(row, 1)], didx)
        pltpu.sync_copy(eac16.at[pl.ds(base, _CH)], eab)
        pltpu.async_copy(as16.at[sidx.at[0]], asb, sem).wait()
        pltpu.async_copy(ad16.at[didx.at[0]], adb, sem).wait()

        @plsc.parallel_loop(0, _CH, 1, unroll=4)
        def _(r):
            l = asb[r, :] + adb[r, :] + eab[r, :]
            l = jnp.maximum(l, 0.2 * l)
            exb[r, :] = jnp.exp(l)

        pltpu.sync_copy(exb, exc16.at[pl.ds(base, _CH)])
        pltpu.sync_copy(exb, den_sh.at[didx.at[0]], add=True)
        return carry

    lax.fori_loop(0, _CPT, chunk, 0)
    plsc.subcore_barrier()
    pltpu.sync_copy(den_sh.at[pl.ds(s * _NPT, _NPT)],
                    den2.at[c, pl.ds(s * _NPT, _NPT)])


def _sca_call(srcc, dstc, as16, ad16, eac16, z16):
    f = pl.kernel(
        _sca_body,
        out_type=(
            jax.ShapeDtypeStruct((E, 16), jnp.float32),
            jax.ShapeDtypeStruct((_SC_NC, N, 16), jnp.float32),
        ),
        mesh=plsc.VectorSubcoreMesh(**_SC_MESH),
        compiler_params=pltpu.CompilerParams(use_tc_tiling_on_sc=False),
        scratch_types=[
            pltpu.VMEM((1, _CH), jnp.int32),
            pltpu.VMEM((1, _CH), jnp.int32),
            pltpu.VMEM((_CH, 16), jnp.float32),
            pltpu.VMEM((_CH, 16), jnp.float32),
            pltpu.VMEM((_CH, 16), jnp.float32),
            pltpu.VMEM((_CH, 16), jnp.float32),
            pltpu.VMEM_SHARED((N, 16), jnp.float32),
            pltpu.SemaphoreType.DMA,
        ],
    )
    return f(srcc, dstc, as16, ad16, eac16, z16)


_UC = 96
_NPASS = NC * HEADS * VAL_C // _UC


def _scb_body(srcc, dstc, exc16, dinv16, vk, z,
              u2, alpha_o,
              smeta, dmeta, dib, exb, ab, vb, u_sh,
              gsem0, gsem1, asem0, asem1):
    c = lax.axis_index("c")
    s = lax.axis_index("s")
    wid = s * _SC_NC + c
    row0 = wid * _CPT
    gsems = (gsem0, gsem1)
    asems = (asem0, asem1)

    pltpu.sync_copy(srcc.at[pl.ds(row0, _CPT)], smeta)
    pltpu.sync_copy(dstc.at[pl.ds(row0, _CPT)], dmeta)

    def alpha_chunk(ch, carry):
        base = (row0 + ch) * _CH
        pltpu.sync_copy(exc16.at[pl.ds(base, _CH)], exb)
        pltpu.async_copy(dinv16.at[dmeta.at[ch]], dib, gsem0).wait()

        @plsc.parallel_loop(0, _CH, 1, unroll=4)
        def _(r):
            exb[r, :] = exb[r, :] * dib[r, :]

        pltpu.sync_copy(exb, alpha_o.at[pl.ds(base, _CH)])
        return carry

    lax.fori_loop(0, _CPT, alpha_chunk, 0)

    for p in range(_NPASS):
        vp = vk.at[p]
        pltpu.sync_copy(z, u_sh.at[pl.ds(s * _NPT, _NPT)])
        plsc.subcore_barrier()

        for b in range(2):
            pltpu.async_copy(vp.at[smeta.at[b]], vb.at[b], gsems[b])
            pltpu.async_copy(alpha_o.at[pl.ds((row0 + b) * _CH, _CH)],
                             ab.at[b], asems[b])

        def pair(g, carry, vp=vp):
            for b in range(2):
                ch = g * 2 + b
                pltpu.make_async_copy(vp.at[smeta.at[0]], vb.at[b],
                                      gsems[b]).wait()
                pltpu.make_async_copy(alpha_o.at[pl.ds(row0 * _CH, _CH)],
                                      ab.at[b], asems[b]).wait()

                @plsc.parallel_loop(0, _CH, 1, unroll=4)
                def _(r):
                    a = ab[b, r, :]
                    for q in range(_UC // 16):
                        sl = pl.ds(q * 16, 16)
                        vb[b, r, sl] = vb[b, r, sl] * a

                pltpu.sync_copy(vb.at[b], u_sh.at[dmeta.at[ch]], add=True)

                @pl.when(ch + 2 < _CPT)
                def _(b=b, ch=ch):
                    pltpu.async_copy(vp.at[smeta.at[ch + 2]], vb.at[b],
                                     gsems[b])
                    pltpu.async_copy(
                        alpha_o.at[pl.ds((row0 + ch + 2) * _CH, _CH)],
                        ab.at[b], asems[b])

            return carry

        lax.fori_loop(0, _CPT // 2, pair, 0)
        plsc.subcore_barrier()
        pltpu.sync_copy(u_sh.at[pl.ds(s * _NPT, _NPT)],
                        u2.at[p, c, pl.ds(s * _NPT, _NPT)])
        plsc.subcore_barrier()


def _scb_call(srcc, dstc, exc16, dinv16, vk, z):
    f = pl.kernel(
        _scb_body,
        out_type=(
            jax.ShapeDtypeStruct((_NPASS, _SC_NC, N, _UC), jnp.float32),
            jax.ShapeDtypeStruct((E, 16), jnp.float32),
        ),
        mesh=plsc.VectorSubcoreMesh(**_SC_MESH),
        compiler_params=pltpu.CompilerParams(use_tc_tiling_on_sc=False),
        scratch_types=[
            pltpu.VMEM((_CPT, _CH), jnp.int32),
            pltpu.VMEM((_CPT, _CH), jnp.int32),
            pltpu.VMEM((_CH, 16), jnp.float32),
            pltpu.VMEM((_CH, 16), jnp.float32),
            pltpu.VMEM((2, _CH, 16), jnp.float32),
            pltpu.VMEM((2, _CH, _UC), jnp.float32),
            pltpu.VMEM_SHARED((N, _UC), jnp.float32),
            pltpu.SemaphoreType.DMA,
            pltpu.SemaphoreType.DMA,
            pltpu.SemaphoreType.DMA,
            pltpu.SemaphoreType.DMA,
        ],
    )
    return f(srcc, dstc, exc16, dinv16, vk, z)


def kernel(bb_rel, bb_features, edge_features, edge_index, noising_mask,
           W_e1, W_alpha, W_v, W_proj, W_g, W_ff, W_eu1, W_eu2):
    src, dst = edge_index[0], edge_index[1]
    mask_f = noising_mask.astype(jnp.float32)

    Wa_s = W_alpha[:TOT_C]
    Wa_d = W_alpha[TOT_C:2 * TOT_C]
    Wa_s16 = jnp.concatenate([Wa_s, Wa_s], axis=1)
    Wa_d16 = jnp.concatenate([Wa_d, Wa_d], axis=1)
    Wv_dh = jnp.swapaxes(W_v, 1, 2).reshape(TOT_C, HEADS * VAL_C)
    vk, as16, ad16 = _node_pre(bb_features, bb_rel, mask_f[:, None],
                               Wv_dh, Wa_s16, Wa_d16)

    Wa_e = W_alpha[2 * TOT_C:]
    Wa_e16 = jnp.concatenate([Wa_e, Wa_e], axis=1)
    Weu_e = W_eu1[2 * BB_C:]
    ea16, e_eu = _edge_pre(edge_features, W_e1, Wa_e16, Weu_e)

    srcc = src.reshape(_NW * _CPT, _CH)
    dstc = dst.reshape(_NW * _CPT, _CH)
    z16 = jnp.zeros((_NPT, 16), jnp.float32)
    exc16, den2 = _sca_call(srcc, dstc, as16, ad16, ea16, z16)
    dinv16 = 1.0 / (den2[0] + den2[1] + 1e-9)

    zuc = jnp.zeros((_NPT, _UC), jnp.float32)
    u2, _ = _scb_call(srcc, dstc, exc16, dinv16, vk, zuc)
    W_proj_dh = (W_proj.reshape(HEADS, VAL_C, BB_C)
                 .transpose(1, 0, 2).reshape(HEADS * VAL_C, BB_C))

    new_bb, b_src, b_dst = _node_post(u2, W_proj_dh, W_g, W_ff,
                                      W_eu1[:BB_C], W_eu1[BB_C:2 * BB_C])

    eeuc = e_eu.reshape(_NW * _CPT, _CH, EDGE_C)
    h_pre = _scd_call(srcc, dstc, b_src, b_dst, eeuc)
    new_edge = _edge_final(h_pre.reshape(E, EDGE_C), W_eu2)
    return new_bb, new_edge


# speedup vs baseline: 20.9621x; 1.1867x over previous
import functools

import jax
import jax.numpy as jnp
from jax import lax
from jax.experimental import pallas as pl
from jax.experimental.pallas import tpu as pltpu
from jax.experimental.pallas import tpu_sc as plsc

N = 10000
E = 160000
NC = 9
BB_C = 32
N_BB = 3
TOT_C = 35
HEADS = 8
VAL_C = 8
EDGE_C = 64

EBLK = 4000


def _edge_pre_kernel(ef_ref, we1_ref, wae_ref, weue_ref, ea_ref, eeu_ref):
    ef = ef_ref[...]
    emb = jax.nn.silu(ef @ we1_ref[...])
    ea_ref[...] = emb @ wae_ref[...]
    eeu_ref[...] = ef @ weue_ref[...]


def _edge_pre(edge_features, W_e1, Wa_e16, Weu_e):
    grid = (E // EBLK,)
    return pl.pallas_call(
        _edge_pre_kernel,
        grid=grid,
        in_specs=[
            pl.BlockSpec((EBLK, EDGE_C), lambda i: (i, 0)),
            pl.BlockSpec((EDGE_C, EDGE_C), lambda i: (0, 0)),
            pl.BlockSpec((EDGE_C, 2 * HEADS), lambda i: (0, 0)),
            pl.BlockSpec((EDGE_C, EDGE_C), lambda i: (0, 0)),
        ],
        out_specs=[
            pl.BlockSpec((EBLK, 2 * HEADS), lambda i: (i, 0)),
            pl.BlockSpec((EBLK, EDGE_C), lambda i: (i, 0)),
        ],
        out_shape=[
            jax.ShapeDtypeStruct((E, 2 * HEADS), jnp.float32),
            jax.ShapeDtypeStruct((E, EDGE_C), jnp.float32),
        ],
    )(edge_features, W_e1, Wa_e16, Weu_e)


def _edge_final_kernel(h_ref, weu2_ref, out_ref):
    h = jax.nn.silu(h_ref[...])
    out_ref[...] = h @ weu2_ref[...]


def _edge_final(h_pre, W_eu2):
    grid = (E // EBLK,)
    return pl.pallas_call(
        _edge_final_kernel,
        grid=grid,
        in_specs=[
            pl.BlockSpec((EBLK, EDGE_C), lambda i: (i, 0)),
            pl.BlockSpec((EDGE_C, EDGE_C), lambda i: (0, 0)),
        ],
        out_specs=pl.BlockSpec((EBLK, EDGE_C), lambda i: (i, 0)),
        out_shape=jax.ShapeDtypeStruct((E, EDGE_C), jnp.float32),
    )(h_pre, W_eu2)


NBLK = 1000


def _node_pre_kernel(bb0_ref, bbf_ref, rel_ref, mask_ref,
                     wv_ref, was_ref, wad_ref,
                     vk_ref, as_ref, ad_ref):
    mask = mask_ref[...]
    zeros2 = jnp.zeros((NBLK, 2), jnp.float32)
    x0 = jnp.concatenate([bb0_ref[...], zeros2, mask], axis=-1)
    as_ref[...] = x0 @ was_ref[...]
    ad_ref[...] = x0 @ wad_ref[...]
    wv = wv_ref[...]
    rel = rel_ref[...]
    vs = []
    for k in range(NC):
        if k == 0:
            xk = x0
        elif 1 <= k <= 3:
            xk = jnp.concatenate(
                [bbf_ref[:, k, :], rel[:, :, k - 1]], axis=-1)
        else:
            xk = jnp.concatenate(
                [bbf_ref[:, k, :], jnp.zeros((NBLK, N_BB), jnp.float32)],
                axis=-1)
        vs.append(xk @ wv)
    vfull = jnp.concatenate(vs, axis=-1)
    for p in range(_NPASS):
        vk_ref[p] = vfull[:, p * _UC:(p + 1) * _UC]


def _node_pre(bb_features, bb_rel, mask16, Wv_dh, Wa_s16, Wa_d16):
    grid = (N // NBLK,)
    return pl.pallas_call(
        _node_pre_kernel,
        grid=grid,
        in_specs=[
            pl.BlockSpec((NBLK, BB_C), lambda i: (i, 0)),
            pl.BlockSpec((NBLK, NC, BB_C), lambda i: (i, 0, 0)),
            pl.BlockSpec((NBLK, N_BB, 3), lambda i: (i, 0, 0)),
            pl.BlockSpec((NBLK, 1), lambda i: (i, 0)),
            pl.BlockSpec((TOT_C, HEADS * VAL_C), lambda i: (0, 0)),
            pl.BlockSpec((TOT_C, 16), lambda i: (0, 0)),
            pl.BlockSpec((TOT_C, 16), lambda i: (0, 0)),
        ],
        out_specs=[
            pl.BlockSpec((_NPASS, NBLK, _UC), lambda i: (0, i, 0)),
            pl.BlockSpec((NBLK, 16), lambda i: (i, 0)),
            pl.BlockSpec((NBLK, 16), lambda i: (i, 0)),
        ],
        out_shape=[
            jax.ShapeDtypeStruct((_NPASS, N, _UC), jnp.float32),
            jax.ShapeDtypeStruct((N, 16), jnp.float32),
            jax.ShapeDtypeStruct((N, 16), jnp.float32),
        ],
    )(bb_features[:, 0, :], bb_features, bb_rel, mask16,
      Wv_dh, Wa_s16, Wa_d16)


def _node_post_kernel(u2_ref, wp_ref, wg_ref, wf_ref, ws_ref, wd_ref,
                      nbb_ref, bs_ref, bd_ref):
    wp = wp_ref[...]
    wf = wf_ref[...]
    ufull = jnp.concatenate(
        [u2_ref[p, 0] + u2_ref[p, 1] for p in range(_NPASS)],
        axis=-1)

    def uk(k):
        return ufull[:, k * 64:(k + 1) * 64]

    agg0 = uk(0) @ wp
    gate = jax.nn.silu(agg0 @ wg_ref[...])
    nb0 = agg0 + (agg0 @ wf) * gate
    nbb_ref[:, 0, :] = nb0
    for k in range(1, NC):
        aggk = uk(k) @ wp
        nbb_ref[:, k, :] = aggk + (aggk @ wf) * gate
    bs_ref[...] = nb0 @ ws_ref[...]
    bd_ref[...] = nb0 @ wd_ref[...]


def _node_post(u2, W_proj_dh, W_g, W_ff, Weu_s, Weu_d):
    grid = (N // NBLK,)
    return pl.pallas_call(
        _node_post_kernel,
        grid=grid,
        in_specs=[
            pl.BlockSpec((_NPASS, _SC_NC, NBLK, _UC),
                         lambda i: (0, 0, i, 0)),
            pl.BlockSpec((HEADS * VAL_C, BB_C), lambda i: (0, 0)),
            pl.BlockSpec((BB_C, BB_C), lambda i: (0, 0)),
            pl.BlockSpec((BB_C, BB_C), lambda i: (0, 0)),
            pl.BlockSpec((BB_C, EDGE_C), lambda i: (0, 0)),
            pl.BlockSpec((BB_C, EDGE_C), lambda i: (0, 0)),
        ],
        out_specs=[
            pl.BlockSpec((NBLK, NC, BB_C), lambda i: (i, 0, 0)),
            pl.BlockSpec((NBLK, EDGE_C), lambda i: (i, 0)),
            pl.BlockSpec((NBLK, EDGE_C), lambda i: (i, 0)),
        ],
        out_shape=[
            jax.ShapeDtypeStruct((N, NC, BB_C), jnp.float32),
            jax.ShapeDtypeStruct((N, EDGE_C), jnp.float32),
            jax.ShapeDtypeStruct((N, EDGE_C), jnp.float32),
        ],
    )(u2, W_proj_dh, W_g, W_ff, Weu_s, Weu_d)


_SC_NC = 2
_SC_NS = 16
_NW = _SC_NC * _SC_NS
_CH = 125
_CPT = E // (_NW * _CH)
_SC_MESH = dict(core_axis_name="c", subcore_axis_name="s")


def _scd_body(srcc, dstc, bsrc, bdst, eeuc, out,
              smeta, dmeta, bs, bd, eb, ob,
              ssem0, ssem1, dsem0, dsem1, esem0, esem1, osem0, osem1):
    c = lax.axis_index("c")
    s = lax.axis_index("s")
    wid = s * _SC_NC + c
    row0 = wid * _CPT
    ssems = (ssem0, ssem1)
    dsems = (dsem0, dsem1)
    esems = (esem0, esem1)
    osems = (osem0, osem1)

    pltpu.sync_copy(srcc.at[pl.ds(row0, _CPT)], smeta)
    pltpu.sync_copy(dstc.at[pl.ds(row0, _CPT)], dmeta)

    for b in range(2):
        pltpu.async_copy(bsrc.at[smeta.at[b]], bs.at[b], ssems[b])
        pltpu.async_copy(bdst.at[dmeta.at[b]], bd.at[b], dsems[b])
        pltpu.async_copy(eeuc.at[pl.ds(row0 + b, 1)], eb.at[b], esems[b])

    def pair(g, carry):
        for b in range(2):
            ch = g * 2 + b
            row = row0 + ch
            pltpu.make_async_copy(bsrc.at[smeta.at[0]], bs.at[b],
                                  ssems[b]).wait()
            pltpu.make_async_copy(bdst.at[dmeta.at[0]], bd.at[b],
                                  dsems[b]).wait()
            pltpu.make_async_copy(eeuc.at[pl.ds(row0, 1)], eb.at[b],
                                  esems[b]).wait()

            @pl.when(ch >= 2)
            def _(b=b):
                pltpu.make_async_copy(ob.at[b], out.at[pl.ds(row0, 1)],
                                      osems[b]).wait()

            @plsc.parallel_loop(0, _CH, 1, unroll=4)
            def _(r):
                for q in range(EDGE_C // 16):
                    sl = pl.ds(q * 16, 16)
                    ob[b, 0, r, sl] = bs[b, r, sl] + bd[b, r, sl] + eb[b, 0, r, sl]

            pltpu.async_copy(ob.at[b], out.at[pl.ds(row, 1)], osems[b])

            @pl.when(ch + 2 < _CPT)
            def _(b=b, ch=ch):
                pltpu.async_copy(bsrc.at[smeta.at[ch + 2]], bs.at[b],
                                 ssems[b])
                pltpu.async_copy(bdst.at[dmeta.at[ch + 2]], bd.at[b],
                                 dsems[b])
                pltpu.async_copy(eeuc.at[pl.ds(row0 + ch + 2, 1)], eb.at[b],
                                 esems[b])

        return carry

    lax.fori_loop(0, _CPT // 2, pair, 0)
    for b in range(2):
        pltpu.make_async_copy(ob.at[b], out.at[pl.ds(row0, 1)],
                              osems[b]).wait()


def _scd_call(srcc, dstc, b_src, b_dst, eeuc):
    f = pl.kernel(
        _scd_body,
        out_type=jax.ShapeDtypeStruct((_NW * _CPT, _CH, EDGE_C), jnp.float32),
        mesh=plsc.VectorSubcoreMesh(**_SC_MESH),
        compiler_params=pltpu.CompilerParams(use_tc_tiling_on_sc=False),
        scratch_types=[
            pltpu.VMEM((_CPT, _CH), jnp.int32),
            pltpu.VMEM((_CPT, _CH), jnp.int32),
            pltpu.VMEM((2, _CH, EDGE_C), jnp.float32),
            pltpu.VMEM((2, _CH, EDGE_C), jnp.float32),
            pltpu.VMEM((2, 1, _CH, EDGE_C), jnp.float32),
            pltpu.VMEM((2, 1, _CH, EDGE_C), jnp.float32),
            pltpu.SemaphoreType.DMA,
            pltpu.SemaphoreType.DMA,
            pltpu.SemaphoreType.DMA,
            pltpu.SemaphoreType.DMA,
            pltpu.SemaphoreType.DMA,
            pltpu.SemaphoreType.DMA,
            pltpu.SemaphoreType.DMA,
            pltpu.SemaphoreType.DMA,
        ],
    )
    return f(srcc, dstc, b_src, b_dst, eeuc)


_NPT = N // _SC_NS


def _sca_body(srcc, dstc, as16, ad16, eac16, z16,
              exc16, den2,
              smeta, dmeta, asb, adb, eab, exb, den_sh,
              ssem0, ssem1, dsem0, dsem1, esem0, esem1, osem0, osem1):
    c = lax.axis_index("c")
    s = lax.axis_index("s")
    wid = s * _SC_NC + c
    row0 = wid * _CPT
    ssems = (ssem0, ssem1)
    dsems = (dsem0, dsem1)
    esems = (esem0, esem1)
    osems = (osem0, osem1)

    pltpu.sync_copy(srcc.at[pl.ds(row0, _CPT)], smeta)
    pltpu.sync_copy(dstc.at[pl.ds(row0, _CPT)], dmeta)
    pltpu.sync_copy(z16, den_sh.at[pl.ds(s * _NPT, _NPT)])
    plsc.subcore_barrier()

    for b in range(2):
        base = (row0 + b) * _CH
        pltpu.async_copy(as16.at[smeta.at[b]], asb.at[b], ssems[b])
        pltpu.async_copy(ad16.at[dmeta.at[b]], adb.at[b], dsems[b])
        pltpu.async_copy(eac16.at[pl.ds(base, _CH)], eab.at[b], esems[b])

    def pair(g, carry):
        for b in range(2):
            ch = g * 2 + b
            base = (row0 + ch) * _CH
            pltpu.make_async_copy(as16.at[smeta.at[0]], asb.at[b],
                                  ssems[b]).wait()
            pltpu.make_async_copy(ad16.at[dmeta.at[0]], adb.at[b],
                                  dsems[b]).wait()
            pltpu.make_async_copy(eac16.at[pl.ds(row0 * _CH, _CH)],
                                  eab.at[b], esems[b]).wait()

            @pl.when(ch >= 2)
            def _(b=b):
                pltpu.make_async_copy(exb.at[b],
                                      exc16.at[pl.ds(row0 * _CH, _CH)],
                                      osems[b]).wait()

            @plsc.parallel_loop(0, _CH, 1, unroll=4)
            def _(r):
                l = asb[b, r, :] + adb[b, r, :] + eab[b, r, :]
                l = jnp.maximum(l, 0.2 * l)
                exb[b, r, :] = jnp.exp(l)

            pltpu.async_copy(exb.at[b], exc16.at[pl.ds(base, _CH)], osems[b])
            pltpu.sync_copy(exb.at[b], den_sh.at[dmeta.at[ch]], add=True)

            @pl.when(ch + 2 < _CPT)
            def _(b=b, ch=ch):
                nbase = (row0 + ch + 2) * _CH
                pltpu.async_copy(as16.at[smeta.at[ch + 2]], asb.at[b],
                                 ssems[b])
                pltpu.async_copy(ad16.at[dmeta.at[ch + 2]], adb.at[b],
                                 dsems[b])
                pltpu.async_copy(eac16.at[pl.ds(nbase, _CH)], eab.at[b],
                                 esems[b])

        return carry

    lax.fori_loop(0, _CPT // 2, pair, 0)
    for b in range(2):
        pltpu.make_async_copy(exb.at[b], exc16.at[pl.ds(row0 * _CH, _CH)],
                              osems[b]).wait()
    plsc.subcore_barrier()
    pltpu.sync_copy(den_sh.at[pl.ds(s * _NPT, _NPT)],
                    den2.at[c, pl.ds(s * _NPT, _NPT)])


def _sca_call(srcc, dstc, as16, ad16, eac16, z16):
    f = pl.kernel(
        _sca_body,
        out_type=(
            jax.ShapeDtypeStruct((E, 16), jnp.float32),
            jax.ShapeDtypeStruct((_SC_NC, N, 16), jnp.float32),
        ),
        mesh=plsc.VectorSubcoreMesh(**_SC_MESH),
        compiler_params=pltpu.CompilerParams(use_tc_tiling_on_sc=False),
        scratch_types=[
            pltpu.VMEM((_CPT, _CH), jnp.int32),
            pltpu.VMEM((_CPT, _CH), jnp.int32),
            pltpu.VMEM((2, _CH, 16), jnp.float32),
            pltpu.VMEM((2, _CH, 16), jnp.float32),
            pltpu.VMEM((2, _CH, 16), jnp.float32),
            pltpu.VMEM((2, _CH, 16), jnp.float32),
            pltpu.VMEM_SHARED((N, 16), jnp.float32),
            pltpu.SemaphoreType.DMA,
            pltpu.SemaphoreType.DMA,
            pltpu.SemaphoreType.DMA,
            pltpu.SemaphoreType.DMA,
            pltpu.SemaphoreType.DMA,
            pltpu.SemaphoreType.DMA,
            pltpu.SemaphoreType.DMA,
            pltpu.SemaphoreType.DMA,
        ],
    )
    return f(srcc, dstc, as16, ad16, eac16, z16)


_UC = 96
_NPASS = NC * HEADS * VAL_C // _UC


def _scb_body(srcc, dstc, exc16, dinv16, vk, z,
              u2, alpha_o,
              smeta, dmeta, dib, exb, ab, vb, u_sh,
              gsem0, gsem1, asem0, asem1):
    c = lax.axis_index("c")
    s = lax.axis_index("s")
    wid = s * _SC_NC + c
    row0 = wid * _CPT
    gsems = (gsem0, gsem1)
    asems = (asem0, asem1)

    pltpu.sync_copy(srcc.at[pl.ds(row0, _CPT)], smeta)
    pltpu.sync_copy(dstc.at[pl.ds(row0, _CPT)], dmeta)

    def alpha_chunk(ch, carry):
        base = (row0 + ch) * _CH
        pltpu.sync_copy(exc16.at[pl.ds(base, _CH)], exb)
        pltpu.async_copy(dinv16.at[dmeta.at[ch]], dib, gsem0).wait()

        @plsc.parallel_loop(0, _CH, 1, unroll=4)
        def _(r):
            exb[r, :] = exb[r, :] * dib[r, :]

        pltpu.sync_copy(exb, alpha_o.at[pl.ds(base, _CH)])
        return carry

    lax.fori_loop(0, _CPT, alpha_chunk, 0)

    for p in range(_NPASS):
        vp = vk.at[p]
        pltpu.sync_copy(z, u_sh.at[pl.ds(s * _NPT, _NPT)])
        plsc.subcore_barrier()

        for b in range(2):
            pltpu.async_copy(vp.at[smeta.at[b]], vb.at[b], gsems[b])
            pltpu.async_copy(alpha_o.at[pl.ds((row0 + b) * _CH, _CH)],
                             ab.at[b], asems[b])

        def pair(g, carry, vp=vp):
            for b in range(2):
                ch = g * 2 + b
                pltpu.make_async_copy(vp.at[smeta.at[0]], vb.at[b],
                                      gsems[b]).wait()
                pltpu.make_async_copy(alpha_o.at[pl.ds(row0 * _CH, _CH)],
                                      ab.at[b], asems[b]).wait()

                @plsc.parallel_loop(0, _CH, 1, unroll=4)
                def _(r):
                    a = ab[b, r, :]
                    for q in range(_UC // 16):
                        sl = pl.ds(q * 16, 16)
                        vb[b, r, sl] = vb[b, r, sl] * a

                pltpu.sync_copy(vb.at[b], u_sh.at[dmeta.at[ch]], add=True)

                @pl.when(ch + 2 < _CPT)
                def _(b=b, ch=ch):
                    pltpu.async_copy(vp.at[smeta.at[ch + 2]], vb.at[b],
                                     gsems[b])
                    pltpu.async_copy(
                        alpha_o.at[pl.ds((row0 + ch + 2) * _CH, _CH)],
                        ab.at[b], asems[b])

            return carry

        lax.fori_loop(0, _CPT // 2, pair, 0)
        plsc.subcore_barrier()
        pltpu.sync_copy(u_sh.at[pl.ds(s * _NPT, _NPT)],
                        u2.at[p, c, pl.ds(s * _NPT, _NPT)])
        plsc.subcore_barrier()


def _scb_call(srcc, dstc, exc16, dinv16, vk, z):
    f = pl.kernel(
        _scb_body,
        out_type=(
            jax.ShapeDtypeStruct((_NPASS, _SC_NC, N, _UC), jnp.float32),
            jax.ShapeDtypeStruct((E, 16), jnp.float32),
        ),
        mesh=plsc.VectorSubcoreMesh(**_SC_MESH),
        compiler_params=pltpu.CompilerParams(use_tc_tiling_on_sc=False),
        scratch_types=[
            pltpu.VMEM((_CPT, _CH), jnp.int32),
            pltpu.VMEM((_CPT, _CH), jnp.int32),
            pltpu.VMEM((_CH, 16), jnp.float32),
            pltpu.VMEM((_CH, 16), jnp.float32),
            pltpu.VMEM((2, _CH, 16), jnp.float32),
            pltpu.VMEM((2, _CH, _UC), jnp.float32),
            pltpu.VMEM_SHARED((N, _UC), jnp.float32),
            pltpu.SemaphoreType.DMA,
            pltpu.SemaphoreType.DMA,
            pltpu.SemaphoreType.DMA,
            pltpu.SemaphoreType.DMA,
        ],
    )
    return f(srcc, dstc, exc16, dinv16, vk, z)


def kernel(bb_rel, bb_features, edge_features, edge_index, noising_mask,
           W_e1, W_alpha, W_v, W_proj, W_g, W_ff, W_eu1, W_eu2):
    src, dst = edge_index[0], edge_index[1]
    mask_f = noising_mask.astype(jnp.float32)

    Wa_s = W_alpha[:TOT_C]
    Wa_d = W_alpha[TOT_C:2 * TOT_C]
    Wa_s16 = jnp.concatenate([Wa_s, Wa_s], axis=1)
    Wa_d16 = jnp.concatenate([Wa_d, Wa_d], axis=1)
    Wv_dh = jnp.swapaxes(W_v, 1, 2).reshape(TOT_C, HEADS * VAL_C)
    vk, as16, ad16 = _node_pre(bb_features, bb_rel, mask_f[:, None],
                               Wv_dh, Wa_s16, Wa_d16)

    Wa_e = W_alpha[2 * TOT_C:]
    Wa_e16 = jnp.concatenate([Wa_e, Wa_e], axis=1)
    Weu_e = W_eu1[2 * BB_C:]
    ea16, e_eu = _edge_pre(edge_features, W_e1, Wa_e16, Weu_e)

    srcc = src.reshape(_NW * _CPT, _CH)
    dstc = dst.reshape(_NW * _CPT, _CH)
    z16 = jnp.zeros((_NPT, 16), jnp.float32)
    exc16, den2 = _sca_call(srcc, dstc, as16, ad16, ea16, z16)
    dinv16 = 1.0 / (den2[0] + den2[1] + 1e-9)

    zuc = jnp.zeros((_NPT, _UC), jnp.float32)
    u2, _ = _scb_call(srcc, dstc, exc16, dinv16, vk, zuc)
    W_proj_dh = (W_proj.reshape(HEADS, VAL_C, BB_C)
                 .transpose(1, 0, 2).reshape(HEADS * VAL_C, BB_C))

    new_bb, b_src, b_dst = _node_post(u2, W_proj_dh, W_g, W_ff,
                                      W_eu1[:BB_C], W_eu1[BB_C:2 * BB_C])

    eeuc = e_eu.reshape(_NW * _CPT, _CH, EDGE_C)
    h_pre = _scd_call(srcc, dstc, b_src, b_dst, eeuc)
    new_edge = _edge_final(h_pre.reshape(E, EDGE_C), W_eu2)
    return new_bb, new_edge


# speedup vs baseline: 21.2903x; 1.0157x over previous
import functools

import jax
import jax.numpy as jnp
from jax import lax
from jax.experimental import pallas as pl
from jax.experimental.pallas import tpu as pltpu
from jax.experimental.pallas import tpu_sc as plsc

N = 10000
E = 160000
NC = 9
BB_C = 32
N_BB = 3
TOT_C = 35
HEADS = 8
VAL_C = 8
EDGE_C = 64

EBLK = 4000


def _edge_pre_kernel(ef_ref, we1_ref, wae_ref, weue_ref, ea_ref, eeu_ref):
    ef = ef_ref[...]
    emb = jax.nn.silu(ef @ we1_ref[...])
    ea_ref[...] = emb @ wae_ref[...]
    eeu_ref[...] = ef @ weue_ref[...]


def _edge_pre(edge_features, W_e1, Wa_e16, Weu_e):
    grid = (E // EBLK,)
    return pl.pallas_call(
        _edge_pre_kernel,
        grid=grid,
        in_specs=[
            pl.BlockSpec((EBLK, EDGE_C), lambda i: (i, 0)),
            pl.BlockSpec((EDGE_C, EDGE_C), lambda i: (0, 0)),
            pl.BlockSpec((EDGE_C, 2 * HEADS), lambda i: (0, 0)),
            pl.BlockSpec((EDGE_C, EDGE_C), lambda i: (0, 0)),
        ],
        out_specs=[
            pl.BlockSpec((EBLK, 2 * HEADS), lambda i: (i, 0)),
            pl.BlockSpec((EBLK, EDGE_C), lambda i: (i, 0)),
        ],
        out_shape=[
            jax.ShapeDtypeStruct((E, 2 * HEADS), jnp.float32),
            jax.ShapeDtypeStruct((E, EDGE_C), jnp.float32),
        ],
    )(edge_features, W_e1, Wa_e16, Weu_e)


def _edge_final_kernel(h_ref, weu2_ref, out_ref):
    h = jax.nn.silu(h_ref[...])
    out_ref[...] = h @ weu2_ref[...]


def _edge_final(h_pre, W_eu2):
    grid = (E // EBLK,)
    return pl.pallas_call(
        _edge_final_kernel,
        grid=grid,
        in_specs=[
            pl.BlockSpec((EBLK, EDGE_C), lambda i: (i, 0)),
            pl.BlockSpec((EDGE_C, EDGE_C), lambda i: (0, 0)),
        ],
        out_specs=pl.BlockSpec((EBLK, EDGE_C), lambda i: (i, 0)),
        out_shape=jax.ShapeDtypeStruct((E, EDGE_C), jnp.float32),
    )(h_pre, W_eu2)


NBLK = 1000


def _node_pre_kernel(bb0_ref, bbf_ref, rel_ref, mask_ref,
                     wv_ref, was_ref, wad_ref,
                     vk_ref, as_ref, ad_ref):
    mask = mask_ref[...]
    zeros2 = jnp.zeros((NBLK, 2), jnp.float32)
    x0 = jnp.concatenate([bb0_ref[...], zeros2, mask], axis=-1)
    as_ref[...] = x0 @ was_ref[...]
    ad_ref[...] = x0 @ wad_ref[...]
    wv = wv_ref[...]
    rel = rel_ref[...]
    vs = []
    for k in range(NC):
        if k == 0:
            xk = x0
        elif 1 <= k <= 3:
            xk = jnp.concatenate(
                [bbf_ref[:, k, :], rel[:, :, k - 1]], axis=-1)
        else:
            xk = jnp.concatenate(
                [bbf_ref[:, k, :], jnp.zeros((NBLK, N_BB), jnp.float32)],
                axis=-1)
        vs.append(xk @ wv)
    vfull = jnp.concatenate(vs, axis=-1)
    for p in range(_NPASS):
        vk_ref[p] = vfull[:, p * _UC:(p + 1) * _UC]


def _node_pre(bb_features, bb_rel, mask16, Wv_dh, Wa_s16, Wa_d16):
    grid = (N // NBLK,)
    return pl.pallas_call(
        _node_pre_kernel,
        grid=grid,
        in_specs=[
            pl.BlockSpec((NBLK, BB_C), lambda i: (i, 0)),
            pl.BlockSpec((NBLK, NC, BB_C), lambda i: (i, 0, 0)),
            pl.BlockSpec((NBLK, N_BB, 3), lambda i: (i, 0, 0)),
            pl.BlockSpec((NBLK, 1), lambda i: (i, 0)),
            pl.BlockSpec((TOT_C, HEADS * VAL_C), lambda i: (0, 0)),
            pl.BlockSpec((TOT_C, 16), lambda i: (0, 0)),
            pl.BlockSpec((TOT_C, 16), lambda i: (0, 0)),
        ],
        out_specs=[
            pl.BlockSpec((_NPASS, NBLK, _UC), lambda i: (0, i, 0)),
            pl.BlockSpec((NBLK, 16), lambda i: (i, 0)),
            pl.BlockSpec((NBLK, 16), lambda i: (i, 0)),
        ],
        out_shape=[
            jax.ShapeDtypeStruct((_NPASS, N, _UC), jnp.float32),
            jax.ShapeDtypeStruct((N, 16), jnp.float32),
            jax.ShapeDtypeStruct((N, 16), jnp.float32),
        ],
    )(bb_features[:, 0, :], bb_features, bb_rel, mask16,
      Wv_dh, Wa_s16, Wa_d16)


def _node_post_kernel(u2_ref, wp_ref, wg_ref, wf_ref, ws_ref, wd_ref,
                      nbb_ref, bs_ref, bd_ref):
    wp = wp_ref[...]
    wf = wf_ref[...]
    ufull = jnp.concatenate(
        [u2_ref[p, 0] + u2_ref[p, 1] for p in range(_NPASS)],
        axis=-1)

    def uk(k):
        return ufull[:, k * 64:(k + 1) * 64]

    agg0 = uk(0) @ wp
    gate = jax.nn.silu(agg0 @ wg_ref[...])
    nb0 = agg0 + (agg0 @ wf) * gate
    nbb_ref[:, 0, :] = nb0
    for k in range(1, NC):
        aggk = uk(k) @ wp
        nbb_ref[:, k, :] = aggk + (aggk @ wf) * gate
    bs_ref[...] = nb0 @ ws_ref[...]
    bd_ref[...] = nb0 @ wd_ref[...]


def _node_post(u2, W_proj_dh, W_g, W_ff, Weu_s, Weu_d):
    grid = (N // NBLK,)
    return pl.pallas_call(
        _node_post_kernel,
        grid=grid,
        in_specs=[
            pl.BlockSpec((_NPASS, _SC_NC, NBLK, _UC),
                         lambda i: (0, 0, i, 0)),
            pl.BlockSpec((HEADS * VAL_C, BB_C), lambda i: (0, 0)),
            pl.BlockSpec((BB_C, BB_C), lambda i: (0, 0)),
            pl.BlockSpec((BB_C, BB_C), lambda i: (0, 0)),
            pl.BlockSpec((BB_C, EDGE_C), lambda i: (0, 0)),
            pl.BlockSpec((BB_C, EDGE_C), lambda i: (0, 0)),
        ],
        out_specs=[
            pl.BlockSpec((NBLK, NC, BB_C), lambda i: (i, 0, 0)),
            pl.BlockSpec((NBLK, EDGE_C), lambda i: (i, 0)),
            pl.BlockSpec((NBLK, EDGE_C), lambda i: (i, 0)),
        ],
        out_shape=[
            jax.ShapeDtypeStruct((N, NC, BB_C), jnp.float32),
            jax.ShapeDtypeStruct((N, EDGE_C), jnp.float32),
            jax.ShapeDtypeStruct((N, EDGE_C), jnp.float32),
        ],
    )(u2, W_proj_dh, W_g, W_ff, Weu_s, Weu_d)


_SC_NC = 2
_SC_NS = 16
_NW = _SC_NC * _SC_NS
_CH = 125
_CPT = E // (_NW * _CH)
_SC_MESH = dict(core_axis_name="c", subcore_axis_name="s")


def _scd_body(srcc, dstc, bsrc, bdst, eeuc, out,
              smeta, dmeta, bs, bd, eb, ob,
              ssem0, ssem1, dsem0, dsem1, esem0, esem1, osem0, osem1):
    c = lax.axis_index("c")
    s = lax.axis_index("s")
    wid = s * _SC_NC + c
    row0 = wid * _CPT
    ssems = (ssem0, ssem1)
    dsems = (dsem0, dsem1)
    esems = (esem0, esem1)
    osems = (osem0, osem1)

    pltpu.sync_copy(srcc.at[pl.ds(row0, _CPT)], smeta)
    pltpu.sync_copy(dstc.at[pl.ds(row0, _CPT)], dmeta)

    for b in range(2):
        pltpu.async_copy(bsrc.at[smeta.at[b]], bs.at[b], ssems[b])
        pltpu.async_copy(bdst.at[dmeta.at[b]], bd.at[b], dsems[b])
        pltpu.async_copy(eeuc.at[pl.ds(row0 + b, 1)], eb.at[b], esems[b])

    def pair(g, carry):
        for b in range(2):
            ch = g * 2 + b
            row = row0 + ch
            pltpu.make_async_copy(bsrc.at[smeta.at[0]], bs.at[b],
                                  ssems[b]).wait()
            pltpu.make_async_copy(bdst.at[dmeta.at[0]], bd.at[b],
                                  dsems[b]).wait()
            pltpu.make_async_copy(eeuc.at[pl.ds(row0, 1)], eb.at[b],
                                  esems[b]).wait()

            @pl.when(ch >= 2)
            def _(b=b):
                pltpu.make_async_copy(ob.at[b], out.at[pl.ds(row0, 1)],
                                      osems[b]).wait()

            @plsc.parallel_loop(0, _CH, 1, unroll=4)
            def _(r):
                for q in range(EDGE_C // 16):
                    sl = pl.ds(q * 16, 16)
                    ob[b, 0, r, sl] = bs[b, r, sl] + bd[b, r, sl] + eb[b, 0, r, sl]

            pltpu.async_copy(ob.at[b], out.at[pl.ds(row, 1)], osems[b])

            @pl.when(ch + 2 < _CPT)
            def _(b=b, ch=ch):
                pltpu.async_copy(bsrc.at[smeta.at[ch + 2]], bs.at[b],
                                 ssems[b])
                pltpu.async_copy(bdst.at[dmeta.at[ch + 2]], bd.at[b],
                                 dsems[b])
                pltpu.async_copy(eeuc.at[pl.ds(row0 + ch + 2, 1)], eb.at[b],
                                 esems[b])

        return carry

    lax.fori_loop(0, _CPT // 2, pair, 0)
    for b in range(2):
        pltpu.make_async_copy(ob.at[b], out.at[pl.ds(row0, 1)],
                              osems[b]).wait()


def _scd_call(srcc, dstc, b_src, b_dst, eeuc):
    f = pl.kernel(
        _scd_body,
        out_type=jax.ShapeDtypeStruct((_NW * _CPT, _CH, EDGE_C), jnp.float32),
        mesh=plsc.VectorSubcoreMesh(**_SC_MESH),
        compiler_params=pltpu.CompilerParams(use_tc_tiling_on_sc=False),
        scratch_types=[
            pltpu.VMEM((_CPT, _CH), jnp.int32),
            pltpu.VMEM((_CPT, _CH), jnp.int32),
            pltpu.VMEM((2, _CH, EDGE_C), jnp.float32),
            pltpu.VMEM((2, _CH, EDGE_C), jnp.float32),
            pltpu.VMEM((2, 1, _CH, EDGE_C), jnp.float32),
            pltpu.VMEM((2, 1, _CH, EDGE_C), jnp.float32),
            pltpu.SemaphoreType.DMA,
            pltpu.SemaphoreType.DMA,
            pltpu.SemaphoreType.DMA,
            pltpu.SemaphoreType.DMA,
            pltpu.SemaphoreType.DMA,
            pltpu.SemaphoreType.DMA,
            pltpu.SemaphoreType.DMA,
            pltpu.SemaphoreType.DMA,
        ],
    )
    return f(srcc, dstc, b_src, b_dst, eeuc)


_NPT = N // _SC_NS


def _sca_body(srcc, dstc, as16, ad16, eac16, z16,
              exc16, den2,
              smeta, dmeta, asb, adb, eab, exb, den_sh,
              ssem0, ssem1, dsem0, dsem1, esem0, esem1, osem0, osem1):
    c = lax.axis_index("c")
    s = lax.axis_index("s")
    wid = s * _SC_NC + c
    row0 = wid * _CPT
    ssems = (ssem0, ssem1)
    dsems = (dsem0, dsem1)
    esems = (esem0, esem1)
    osems = (osem0, osem1)

    pltpu.sync_copy(srcc.at[pl.ds(row0, _CPT)], smeta)
    pltpu.sync_copy(dstc.at[pl.ds(row0, _CPT)], dmeta)
    pltpu.sync_copy(z16, den_sh.at[pl.ds(s * _NPT, _NPT)])
    plsc.subcore_barrier()

    for b in range(2):
        base = (row0 + b) * _CH
        pltpu.async_copy(as16.at[smeta.at[b]], asb.at[b], ssems[b])
        pltpu.async_copy(ad16.at[dmeta.at[b]], adb.at[b], dsems[b])
        pltpu.async_copy(eac16.at[pl.ds(base, _CH)], eab.at[b], esems[b])

    def pair(g, carry):
        for b in range(2):
            ch = g * 2 + b
            base = (row0 + ch) * _CH
            pltpu.make_async_copy(as16.at[smeta.at[0]], asb.at[b],
                                  ssems[b]).wait()
            pltpu.make_async_copy(ad16.at[dmeta.at[0]], adb.at[b],
                                  dsems[b]).wait()
            pltpu.make_async_copy(eac16.at[pl.ds(row0 * _CH, _CH)],
                                  eab.at[b], esems[b]).wait()

            @pl.when(ch >= 2)
            def _(b=b):
                pltpu.make_async_copy(exb.at[b],
                                      exc16.at[pl.ds(row0 * _CH, _CH)],
                                      osems[b]).wait()

            @plsc.parallel_loop(0, _CH, 1, unroll=4)
            def _(r):
                l = asb[b, r, :] + adb[b, r, :] + eab[b, r, :]
                l = jnp.maximum(l, 0.2 * l)
                exb[b, r, :] = jnp.exp(l)

            pltpu.async_copy(exb.at[b], exc16.at[pl.ds(base, _CH)], osems[b])
            pltpu.sync_copy(exb.at[b], den_sh.at[dmeta.at[ch]], add=True)

            @pl.when(ch + 2 < _CPT)
            def _(b=b, ch=ch):
                nbase = (row0 + ch + 2) * _CH
                pltpu.async_copy(as16.at[smeta.at[ch + 2]], asb.at[b],
                                 ssems[b])
                pltpu.async_copy(ad16.at[dmeta.at[ch + 2]], adb.at[b],
                                 dsems[b])
                pltpu.async_copy(eac16.at[pl.ds(nbase, _CH)], eab.at[b],
                                 esems[b])

        return carry

    lax.fori_loop(0, _CPT // 2, pair, 0)
    for b in range(2):
        pltpu.make_async_copy(exb.at[b], exc16.at[pl.ds(row0 * _CH, _CH)],
                              osems[b]).wait()
    plsc.subcore_barrier()
    pltpu.sync_copy(den_sh.at[pl.ds(s * _NPT, _NPT)],
                    den2.at[c, pl.ds(s * _NPT, _NPT)])


def _sca_call(srcc, dstc, as16, ad16, eac16, z16):
    f = pl.kernel(
        _sca_body,
        out_type=(
            jax.ShapeDtypeStruct((E, 16), jnp.float32),
            jax.ShapeDtypeStruct((_SC_NC, N, 16), jnp.float32),
        ),
        mesh=plsc.VectorSubcoreMesh(**_SC_MESH),
        compiler_params=pltpu.CompilerParams(use_tc_tiling_on_sc=False),
        scratch_types=[
            pltpu.VMEM((_CPT, _CH), jnp.int32),
            pltpu.VMEM((_CPT, _CH), jnp.int32),
            pltpu.VMEM((2, _CH, 16), jnp.float32),
            pltpu.VMEM((2, _CH, 16), jnp.float32),
            pltpu.VMEM((2, _CH, 16), jnp.float32),
            pltpu.VMEM((2, _CH, 16), jnp.float32),
            pltpu.VMEM_SHARED((N, 16), jnp.float32),
            pltpu.SemaphoreType.DMA,
            pltpu.SemaphoreType.DMA,
            pltpu.SemaphoreType.DMA,
            pltpu.SemaphoreType.DMA,
            pltpu.SemaphoreType.DMA,
            pltpu.SemaphoreType.DMA,
            pltpu.SemaphoreType.DMA,
            pltpu.SemaphoreType.DMA,
        ],
    )
    return f(srcc, dstc, as16, ad16, eac16, z16)


_UC = 96
_NPASS = NC * HEADS * VAL_C // _UC


def _scb_body(srcc, dstc, exc16, dinv16, vk,
              u2, alpha_o,
              smeta, dmeta, dib, exb, ab, vb, zb, u_sh,
              gsem0, gsem1, asem0, asem1):
    c = lax.axis_index("c")
    s = lax.axis_index("s")
    wid = s * _SC_NC + c
    row0 = wid * _CPT
    gsems = (gsem0, gsem1)
    asems = (asem0, asem1)

    pltpu.sync_copy(srcc.at[pl.ds(row0, _CPT)], smeta)
    pltpu.sync_copy(dstc.at[pl.ds(row0, _CPT)], dmeta)

    def alpha_chunk(ch, carry):
        base = (row0 + ch) * _CH
        pltpu.sync_copy(exc16.at[pl.ds(base, _CH)], exb)
        pltpu.async_copy(dinv16.at[dmeta.at[ch]], dib, gsem0).wait()

        @plsc.parallel_loop(0, _CH, 1, unroll=4)
        def _(r):
            exb[r, :] = exb[r, :] * dib[r, :]

        pltpu.sync_copy(exb, alpha_o.at[pl.ds(base, _CH)])
        return carry

    lax.fori_loop(0, _CPT, alpha_chunk, 0)

    zero = jnp.zeros((16,), jnp.float32)

    @plsc.parallel_loop(0, _CH, 1, unroll=8)
    def _(r):
        for q in range(_UC // 16):
            zb[r, pl.ds(q * 16, 16)] = zero

    for p in range(_NPASS):
        vp = vk.at[p]
        for t in range(_NPT // _CH):
            pltpu.sync_copy(zb, u_sh.at[pl.ds(s * _NPT + t * _CH, _CH)])
        plsc.subcore_barrier()

        for b in range(2):
            pltpu.async_copy(vp.at[smeta.at[b]], vb.at[b], gsems[b])
            pltpu.async_copy(alpha_o.at[pl.ds((row0 + b) * _CH, _CH)],
                             ab.at[b], asems[b])

        def pair(g, carry, vp=vp):
            for b in range(2):
                ch = g * 2 + b
                pltpu.make_async_copy(vp.at[smeta.at[0]], vb.at[b],
                                      gsems[b]).wait()
                pltpu.make_async_copy(alpha_o.at[pl.ds(row0 * _CH, _CH)],
                                      ab.at[b], asems[b]).wait()

                @plsc.parallel_loop(0, _CH, 1, unroll=4)
                def _(r):
                    a = ab[b, r, :]
                    for q in range(_UC // 16):
                        sl = pl.ds(q * 16, 16)
                        vb[b, r, sl] = vb[b, r, sl] * a

                pltpu.sync_copy(vb.at[b], u_sh.at[dmeta.at[ch]], add=True)

                @pl.when(ch + 2 < _CPT)
                def _(b=b, ch=ch):
                    pltpu.async_copy(vp.at[smeta.at[ch + 2]], vb.at[b],
                                     gsems[b])
                    pltpu.async_copy(
                        alpha_o.at[pl.ds((row0 + ch + 2) * _CH, _CH)],
                        ab.at[b], asems[b])

            return carry

        lax.fori_loop(0, _CPT // 2, pair, 0)
        plsc.subcore_barrier()
        pltpu.sync_copy(u_sh.at[pl.ds(s * _NPT, _NPT)],
                        u2.at[p, c, pl.ds(s * _NPT, _NPT)])
        plsc.subcore_barrier()


def _scb_call(srcc, dstc, exc16, dinv16, vk):
    f = pl.kernel(
        _scb_body,
        out_type=(
            jax.ShapeDtypeStruct((_NPASS, _SC_NC, N, _UC), jnp.float32),
            jax.ShapeDtypeStruct((E, 16), jnp.float32),
        ),
        mesh=plsc.VectorSubcoreMesh(**_SC_MESH),
        compiler_params=pltpu.CompilerParams(use_tc_tiling_on_sc=False),
        scratch_types=[
            pltpu.VMEM((_CPT, _CH), jnp.int32),
            pltpu.VMEM((_CPT, _CH), jnp.int32),
            pltpu.VMEM((_CH, 16), jnp.float32),
            pltpu.VMEM((_CH, 16), jnp.float32),
            pltpu.VMEM((2, _CH, 16), jnp.float32),
            pltpu.VMEM((2, _CH, _UC), jnp.float32),
            pltpu.VMEM((_CH, _UC), jnp.float32),
            pltpu.VMEM_SHARED((N, _UC), jnp.float32),
            pltpu.SemaphoreType.DMA,
            pltpu.SemaphoreType.DMA,
            pltpu.SemaphoreType.DMA,
            pltpu.SemaphoreType.DMA,
        ],
    )
    return f(srcc, dstc, exc16, dinv16, vk)


def kernel(bb_rel, bb_features, edge_features, edge_index, noising_mask,
           W_e1, W_alpha, W_v, W_proj, W_g, W_ff, W_eu1, W_eu2):
    src, dst = edge_index[0], edge_index[1]
    mask_f = noising_mask.astype(jnp.float32)

    Wa_s = W_alpha[:TOT_C]
    Wa_d = W_alpha[TOT_C:2 * TOT_C]
    Wa_s16 = jnp.concatenate([Wa_s, Wa_s], axis=1)
    Wa_d16 = jnp.concatenate([Wa_d, Wa_d], axis=1)
    Wv_dh = jnp.swapaxes(W_v, 1, 2).reshape(TOT_C, HEADS * VAL_C)
    vk, as16, ad16 = _node_pre(bb_features, bb_rel, mask_f[:, None],
                               Wv_dh, Wa_s16, Wa_d16)

    Wa_e = W_alpha[2 * TOT_C:]
    Wa_e16 = jnp.concatenate([Wa_e, Wa_e], axis=1)
    Weu_e = W_eu1[2 * BB_C:]
    ea16, e_eu = _edge_pre(edge_features, W_e1, Wa_e16, Weu_e)

    srcc = src.reshape(_NW * _CPT, _CH)
    dstc = dst.reshape(_NW * _CPT, _CH)
    z16 = jnp.zeros((_NPT, 16), jnp.float32)
    exc16, den2 = _sca_call(srcc, dstc, as16, ad16, ea16, z16)
    dinv16 = 1.0 / (den2[0] + den2[1] + 1e-9)

    eeuc = e_eu.reshape(_NW * _CPT, _CH, EDGE_C)
    u2, _ = _scb_call(srcc, dstc, exc16, dinv16, vk)
    W_proj_dh = (W_proj.reshape(HEADS, VAL_C, BB_C)
                 .transpose(1, 0, 2).reshape(HEADS * VAL_C, BB_C))

    new_bb, b_src, b_dst = _node_post(u2, W_proj_dh, W_g, W_ff,
                                      W_eu1[:BB_C], W_eu1[BB_C:2 * BB_C])

    h_pre = _scd_call(srcc, dstc, b_src, b_dst, eeuc)
    new_edge = _edge_final(h_pre.reshape(E, EDGE_C), W_eu2)
    return new_bb, new_edge
